# baseline (speedup 1.0000x reference)
import functools
import math

import jax
import jax.numpy as jnp
from jax import lax
from jax.experimental import pallas as pl
from jax.experimental.pallas import tpu as pltpu
from jax.experimental.pallas import tpu_sc as plsc

N = 50000
M = 25000
E = 800000
FIN = 128
D = 64
HD = 32

NC = 2
NS = 16

CW = 128
CH = 392
PAIRS_PAD = NS * CH * CW
PADP = PAIRS_PAD - E
IROWS = PAIRS_PAD // CW

NPT = 50176
TROWS = NC * NPT
AROWS = 26624
IB = 56
VH = AROWS
MC = 25088
ZBLK = 97
ZROW = ZBLK * 512

BN = 512
NBLK = NPT // BN
HBLK = AROWS // BN

_f32 = jnp.float32
_i32 = jnp.int32

PER_TILE = AROWS // NS


def _fill_vmem(ref, rows, cols, value):
    def body(r, carry):
        for j in range(cols // 16):
            ref[r, pl.ds(16 * j, 16)] = jnp.full((16,), value, _f32)
        return carry
    lax.fori_loop(0, rows, body, 0)


@functools.partial(
    pl.kernel,
    out_type=jax.ShapeDtypeStruct((NC * AROWS, HD), _f32),
    mesh=plsc.VectorSubcoreMesh(core_axis_name="c", subcore_axis_name="s"),
    scratch_types=[
        pltpu.VMEM((IB, CW), _i32),
        pltpu.VMEM((IB, CW), _i32),
        pltpu.VMEM((CW, HD), _f32),
        pltpu.VMEM((CW, HD), _f32),
        pltpu.VMEM_SHARED((AROWS, HD), _f32),
        pltpu.SemaphoreType.DMA,
    ],
    compiler_params=pltpu.CompilerParams(use_tc_tiling_on_sc=False,
                                         has_side_effects=True),
)
def _sc_pass(table, gidx_hbm, sidx_hbm, out, gidx, sidx, rows, zbuf, acc, sem):
    c = lax.axis_index("c")
    s = lax.axis_index("s")
    _fill_vmem(zbuf, CW, HD, 0.0)
    def zacc(k, carry):
        pltpu.sync_copy(zbuf, acc.at[pl.ds(s * PER_TILE + k * CW, CW)])
        return carry
    lax.fori_loop(0, PER_TILE // CW, zacc, 0)
    plsc.subcore_barrier()

    def superchunk(u, carry):
        pltpu.sync_copy(gidx_hbm.at[pl.ds((c * NS + s) * CH + u * IB, IB)],
                        gidx)
        pltpu.sync_copy(sidx_hbm.at[pl.ds(s * CH + u * IB, IB)], sidx)
        def chunk(i, carry2):
            pltpu.async_copy(table.at[gidx.at[i]], rows, sem).wait()
            pltpu.sync_copy(rows, acc.at[sidx.at[i]], add=True)
            return carry2
        lax.fori_loop(0, IB, chunk, 0)
        return carry

    lax.fori_loop(0, CH // IB, superchunk, 0)
    plsc.subcore_barrier()
    pltpu.sync_copy(acc.at[pl.ds(s * PER_TILE, PER_TILE)],
                    out.at[pl.ds(c * AROWS + s * PER_TILE, PER_TILE)])


def _dense_in_body(x_ref, w_ref, b_ref, x0_ref, xt_ref):
    xb = x_ref[...]
    xw = lax.dot_general(xb, w_ref[...], (((1,), (1,)), ((), ())),
                         preferred_element_type=_f32)
    xw = jnp.maximum(xw + b_ref[...][0], 0.0)
    x0_ref[...] = xw
    xt_ref[0] = xw[:, :HD]
    xt_ref[1] = xw[:, HD:]


def _dense_in(x, w0, b0):
    return pl.pallas_call(
        _dense_in_body,
        grid=(NBLK,),
        in_specs=[
            pl.BlockSpec((BN, FIN), lambda i: (i, 0)),
            pl.BlockSpec((D, FIN), lambda i: (0, 0)),
            pl.BlockSpec((8, D), lambda i: (0, 0)),
        ],
        out_specs=[
            pl.BlockSpec((BN, D), lambda i: (i, 0)),
            pl.BlockSpec((2, BN, HD), lambda i: (0, i, 0)),
        ],
        out_shape=[
            jax.ShapeDtypeStruct((NPT, D), _f32),
            jax.ShapeDtypeStruct((2, NPT, HD), _f32),
        ],
    )(x, w0, b0)


def _mk_xe_body(pa_ref, cnt_ref, dege_ref, out_ref):
    j = pl.program_id(0) % 50

    @pl.when(j < 49)
    def _():
        cnt = cnt_ref[...][:, 0:1]
        scale = dege_ref[...] / jnp.maximum(cnt, 1.0)
        out_ref[...] = pa_ref[...] * scale

    @pl.when(j == 49)
    def _():
        out_ref[...] = jnp.zeros((BN, HD), _f32)


def _mk_xe(pa, cnt, dege):
    return pl.pallas_call(
        _mk_xe_body,
        grid=(NC * 50,),
        in_specs=[
            pl.BlockSpec(
                (BN, HD),
                lambda i: (HBLK * (i // 50) + jnp.minimum(i % 50, 48), 0)),
            pl.BlockSpec((BN, 16), lambda i: (jnp.minimum(i % 50, 48), 0)),
            pl.BlockSpec((BN, 1), lambda i: (jnp.minimum(i % 50, 48), 0)),
        ],
        out_specs=pl.BlockSpec(
            (BN, HD),
            lambda i: (NBLK * (i // 50) + jnp.where(i % 50 < 49, i % 50, ZBLK),
                       0)),
        out_shape=jax.ShapeDtypeStruct((TROWS, HD), _f32),
    )(pa, cnt, dege)


def _make_layer_fin(beta):
    def body(h0lo_ref, h0hi_ref, h1lo_ref, h1hi_ref, degv_ref, x0_ref, wc_ref,
             out_ref):
        use0 = pl.program_id(0) < HBLK
        xlo = jnp.where(use0, h0lo_ref[...], h1lo_ref[...])
        xhi = jnp.where(use0, h0hi_ref[...], h1hi_ref[...])
        xv = jnp.concatenate([xlo, xhi], axis=1)
        xv = xv * degv_ref[...]
        nrm = jnp.sqrt(jnp.sum(xv * xv, axis=1, keepdims=True))
        scale = jnp.where(nrm > 0, 1.0 / jnp.maximum(nrm, 1e-30), 0.0)
        xi = 0.9 * (xv * scale) + 0.1 * x0_ref[...]
        xw = lax.dot_general(xi, wc_ref[...], (((1,), (1,)), ((), ())),
                             preferred_element_type=_f32)
        xl = jnp.maximum((1.0 - beta) * xi + beta * xw, 0.0)
        out_ref[0] = xl[:, :HD]
        out_ref[1] = xl[:, HD:]

    def run(pa0, pa1, degv, x0, wc):
        h1b = lambda i: jnp.clip(i - HBLK, 0, 45)
        return pl.pallas_call(
            body,
            grid=(NBLK,),
            in_specs=[
                pl.BlockSpec((BN, HD), lambda i: (jnp.minimum(i, HBLK - 1), 0)),
                pl.BlockSpec((BN, HD),
                             lambda i: (HBLK + jnp.minimum(i, HBLK - 1), 0)),
                pl.BlockSpec((BN, HD), lambda i: (h1b(i), 0)),
                pl.BlockSpec((BN, HD), lambda i: (HBLK + h1b(i), 0)),
                pl.BlockSpec((BN, 1), lambda i: (i, 0)),
                pl.BlockSpec((BN, D), lambda i: (i, 0)),
                pl.BlockSpec((D, D), lambda i: (0, 0)),
            ],
            out_specs=pl.BlockSpec((2, BN, HD), lambda i: (0, i, 0)),
            out_shape=jax.ShapeDtypeStruct((2, NPT, HD), _f32),
        )(pa0, pa0, pa1, pa1, degv, x0, wc)

    return run


_layer_fin = [_make_layer_fin(math.log(0.5 / (i + 1) + 1.0)) for i in range(2)]


def _dense_out_body(lo_ref, hi_ref, w_ref, b_ref, out_ref):
    xb = jnp.concatenate([lo_ref[0], hi_ref[0]], axis=1)
    xw = lax.dot_general(xb, w_ref[...], (((1,), (1,)), ((), ())),
                         preferred_element_type=_f32)
    out_ref[...] = xw + b_ref[...][0]


def _dense_out(xt, wout, bout):
    return pl.pallas_call(
        _dense_out_body,
        grid=(NBLK,),
        in_specs=[
            pl.BlockSpec((1, BN, HD), lambda i: (0, i, 0)),
            pl.BlockSpec((1, BN, HD), lambda i: (1, i, 0)),
            pl.BlockSpec((FIN, D), lambda i: (0, 0)),
            pl.BlockSpec((8, FIN), lambda i: (0, 0)),
        ],
        out_specs=pl.BlockSpec((BN, FIN), lambda i: (i, 0)),
        out_shape=jax.ShapeDtypeStruct((NPT, FIN), _f32),
    )(xt, xt, wout, bout)


def kernel(x, H, vertex, edges, degV, degE, W0, b0, Wc, Wout, bout):
    vertex = vertex.astype(_i32)
    edges = edges.astype(_i32)
    zpad = jnp.zeros((PADP,), _i32)
    vg = jnp.concatenate([vertex, zpad])
    vgi = jnp.concatenate([vg, vg + NPT]).reshape(NC * IROWS, CW)
    esi = jnp.concatenate([edges, jnp.full((PADP,), M, _i32)]).reshape(IROWS, CW)

    vm = jnp.concatenate([vertex, jnp.full((PADP,), jnp.int32(1 << 30))])
    ep = jnp.concatenate([edges, zpad])
    egi_h = []
    vsi_h = []
    for h in range(2):
        mask = (vm >= h * VH) & (vm < (h + 1) * VH)
        eg = jnp.where(mask, ep, ZROW)
        egi_h.append(jnp.concatenate([eg, eg + NPT]).reshape(NC * IROWS, CW))
        vsi_h.append(jnp.where(mask, vm - h * VH, 0).reshape(IROWS, CW))

    b0r = jnp.broadcast_to(b0.reshape(1, D), (8, D))
    boutr = jnp.broadcast_to(bout.reshape(1, FIN), (8, FIN))

    x0, xt = _dense_in(x, W0, b0r)
    xt0_flat = xt.reshape(TROWS, HD)

    gsel = jnp.array([0, 1, 2, 3, 1, 2, 3], _i32)
    ssel = jnp.array([0, 0, 1, 2, 0, 1, 2], _i32)
    brs = jnp.array([0, 1, 2, 3, 1, 2, 4], _i32)
    gidx_all = jnp.stack([jnp.zeros((NC * IROWS, CW), _i32), vgi,
                          egi_h[0], egi_h[1]])
    sidx_all = jnp.stack([esi, vsi_h[0], vsi_h[1]])
    ones_tab = jnp.ones((TROWS, HD), _f32)

    def loop_body(t, state):
        table, cnt, prev = state
        gidx = lax.dynamic_index_in_dim(gidx_all, gsel[t], 0, keepdims=False)
        sidx = lax.dynamic_index_in_dim(sidx_all, ssel[t], 0, keepdims=False)
        pa = _sc_pass(table, gidx, sidx)

        def b_counts(_):
            return xt0_flat, lax.slice(pa, (0, 0), (MC, 16)), pa

        def b_mk_xe(_):
            return _mk_xe(pa, cnt, degE), cnt, pa

        def b_stash(_):
            return table, cnt, pa

        def b_fin0(_):
            return (_layer_fin[0](prev, pa, degV, x0, Wc[0]).reshape(TROWS, HD),
                    cnt, pa)

        def b_fin1(_):
            return (_layer_fin[1](prev, pa, degV, x0, Wc[1]).reshape(TROWS, HD),
                    cnt, pa)

        return lax.switch(brs[t], [b_counts, b_mk_xe, b_stash, b_fin0, b_fin1],
                          0)

    cnt0 = jnp.zeros((MC, 16), _f32)
    prev0 = jnp.zeros((NC * AROWS, HD), _f32)
    table_fin, _, _ = lax.fori_loop(0, 7, loop_body, (ones_tab, cnt0, prev0))

    out = _dense_out(table_fin.reshape(NC, NPT, HD), Wout, boutr)
    return out[:N]


# speedup vs baseline: 1.0092x; 1.0092x over previous
import functools
import math

import jax
import jax.numpy as jnp
from jax import lax
from jax.experimental import pallas as pl
from jax.experimental.pallas import tpu as pltpu
from jax.experimental.pallas import tpu_sc as plsc

N = 50000
M = 25000
E = 800000
FIN = 128
D = 64
HD = 32

NC = 2
NS = 16

CW = 128
CH = 392
PAIRS_PAD = NS * CH * CW
PADP = PAIRS_PAD - E
IROWS = PAIRS_PAD // CW

NPT = 50176
TROWS = NC * NPT
AROWS = 26624
IB = 8
NBUF = 4
VH = AROWS
MC = 25088
ZBLK = 97
ZROW = ZBLK * 512

BN = 512
NBLK = NPT // BN
HBLK = AROWS // BN

_f32 = jnp.float32
_i32 = jnp.int32

PER_TILE = AROWS // NS


def _fill_vmem(ref, rows, cols, value):
    def body(r, carry):
        for j in range(cols // 16):
            ref[r, pl.ds(16 * j, 16)] = jnp.full((16,), value, _f32)
        return carry
    lax.fori_loop(0, rows, body, 0)


@functools.partial(
    pl.kernel,
    out_type=jax.ShapeDtypeStruct((NC * AROWS, HD), _f32),
    mesh=plsc.VectorSubcoreMesh(core_axis_name="c", subcore_axis_name="s"),
    scratch_types=[
        pltpu.VMEM((IB, CW), _i32),
        pltpu.VMEM((IB, CW), _i32),
        pltpu.VMEM((CW, HD), _f32),
        pltpu.VMEM((CW, HD), _f32),
        pltpu.VMEM((CW, HD), _f32),
        pltpu.VMEM((CW, HD), _f32),
        pltpu.VMEM_SHARED((AROWS, HD), _f32),
        pltpu.SemaphoreType.DMA,
        pltpu.SemaphoreType.DMA,
    ],
    compiler_params=pltpu.CompilerParams(use_tc_tiling_on_sc=False,
                                         has_side_effects=True),
)
def _sc_pass(table, gidx_hbm, sidx_hbm, out, gidx, sidx, r0, r1, r2, r3, acc,
             semg, sems):
    c = lax.axis_index("c")
    s = lax.axis_index("s")
    rows = [r0, r1, r2, r3]
    _fill_vmem(r0, CW, HD, 0.0)
    def zacc(k, carry):
        pltpu.sync_copy(r0, acc.at[pl.ds(s * PER_TILE + k * CW, CW)])
        return carry
    lax.fori_loop(0, PER_TILE // CW, zacc, 0)
    plsc.subcore_barrier()

    def superchunk(u, carry):
        pltpu.sync_copy(gidx_hbm.at[pl.ds((c * NS + s) * CH + u * IB, IB)],
                        gidx)
        pltpu.sync_copy(sidx_hbm.at[pl.ds(s * CH + u * IB, IB)], sidx)
        for g in range(IB // NBUF):
            gets = [pltpu.async_copy(table.at[gidx.at[g * NBUF + b]], rows[b],
                                     semg) for b in range(NBUF)]
            for d in gets:
                d.wait()
            puts = [pltpu.async_copy(rows[b], acc.at[sidx.at[g * NBUF + b]],
                                     sems, add=True) for b in range(NBUF)]
            for d in puts:
                d.wait()
        return carry

    lax.fori_loop(0, CH // IB, superchunk, 0)
    plsc.subcore_barrier()
    pltpu.sync_copy(acc.at[pl.ds(s * PER_TILE, PER_TILE)],
                    out.at[pl.ds(c * AROWS + s * PER_TILE, PER_TILE)])


def _dense_in_body(x_ref, w_ref, b_ref, x0_ref, xt_ref):
    xb = x_ref[...]
    xw = lax.dot_general(xb, w_ref[...], (((1,), (1,)), ((), ())),
                         preferred_element_type=_f32)
    xw = jnp.maximum(xw + b_ref[...][0], 0.0)
    x0_ref[...] = xw
    xt_ref[0] = xw[:, :HD]
    xt_ref[1] = xw[:, HD:]


def _dense_in(x, w0, b0):
    return pl.pallas_call(
        _dense_in_body,
        grid=(NBLK,),
        in_specs=[
            pl.BlockSpec((BN, FIN), lambda i: (i, 0)),
            pl.BlockSpec((D, FIN), lambda i: (0, 0)),
            pl.BlockSpec((8, D), lambda i: (0, 0)),
        ],
        out_specs=[
            pl.BlockSpec((BN, D), lambda i: (i, 0)),
            pl.BlockSpec((2, BN, HD), lambda i: (0, i, 0)),
        ],
        out_shape=[
            jax.ShapeDtypeStruct((NPT, D), _f32),
            jax.ShapeDtypeStruct((2, NPT, HD), _f32),
        ],
    )(x, w0, b0)


def _mk_xe_body(pa_ref, cnt_ref, dege_ref, out_ref):
    j = pl.program_id(0) % 50

    @pl.when(j < 49)
    def _():
        cnt = cnt_ref[...][:, 0:1]
        scale = dege_ref[...] / jnp.maximum(cnt, 1.0)
        out_ref[...] = pa_ref[...] * scale

    @pl.when(j == 49)
    def _():
        out_ref[...] = jnp.zeros((BN, HD), _f32)


def _mk_xe(pa, cnt, dege):
    return pl.pallas_call(
        _mk_xe_body,
        grid=(NC * 50,),
        in_specs=[
            pl.BlockSpec(
                (BN, HD),
                lambda i: (HBLK * (i // 50) + jnp.minimum(i % 50, 48), 0)),
            pl.BlockSpec((BN, 16), lambda i: (jnp.minimum(i % 50, 48), 0)),
            pl.BlockSpec((BN, 1), lambda i: (jnp.minimum(i % 50, 48), 0)),
        ],
        out_specs=pl.BlockSpec(
            (BN, HD),
            lambda i: (NBLK * (i // 50) + jnp.where(i % 50 < 49, i % 50, ZBLK),
                       0)),
        out_shape=jax.ShapeDtypeStruct((TROWS, HD), _f32),
    )(pa, cnt, dege)


def _make_layer_fin(beta):
    def body(h0lo_ref, h0hi_ref, h1lo_ref, h1hi_ref, degv_ref, x0_ref, wc_ref,
             out_ref):
        use0 = pl.program_id(0) < HBLK
        xlo = jnp.where(use0, h0lo_ref[...], h1lo_ref[...])
        xhi = jnp.where(use0, h0hi_ref[...], h1hi_ref[...])
        xv = jnp.concatenate([xlo, xhi], axis=1)
        xv = xv * degv_ref[...]
        nrm = jnp.sqrt(jnp.sum(xv * xv, axis=1, keepdims=True))
        scale = jnp.where(nrm > 0, 1.0 / jnp.maximum(nrm, 1e-30), 0.0)
        xi = 0.9 * (xv * scale) + 0.1 * x0_ref[...]
        xw = lax.dot_general(xi, wc_ref[...], (((1,), (1,)), ((), ())),
                             preferred_element_type=_f32)
        xl = jnp.maximum((1.0 - beta) * xi + beta * xw, 0.0)
        out_ref[0] = xl[:, :HD]
        out_ref[1] = xl[:, HD:]

    def run(pa0, pa1, degv, x0, wc):
        h1b = lambda i: jnp.clip(i - HBLK, 0, 45)
        return pl.pallas_call(
            body,
            grid=(NBLK,),
            in_specs=[
                pl.BlockSpec((BN, HD), lambda i: (jnp.minimum(i, HBLK - 1), 0)),
                pl.BlockSpec((BN, HD),
                             lambda i: (HBLK + jnp.minimum(i, HBLK - 1), 0)),
                pl.BlockSpec((BN, HD), lambda i: (h1b(i), 0)),
                pl.BlockSpec((BN, HD), lambda i: (HBLK + h1b(i), 0)),
                pl.BlockSpec((BN, 1), lambda i: (i, 0)),
                pl.BlockSpec((BN, D), lambda i: (i, 0)),
                pl.BlockSpec((D, D), lambda i: (0, 0)),
            ],
            out_specs=pl.BlockSpec((2, BN, HD), lambda i: (0, i, 0)),
            out_shape=jax.ShapeDtypeStruct((2, NPT, HD), _f32),
        )(pa0, pa0, pa1, pa1, degv, x0, wc)

    return run


_layer_fin = [_make_layer_fin(math.log(0.5 / (i + 1) + 1.0)) for i in range(2)]


def _dense_out_body(lo_ref, hi_ref, w_ref, b_ref, out_ref):
    xb = jnp.concatenate([lo_ref[0], hi_ref[0]], axis=1)
    xw = lax.dot_general(xb, w_ref[...], (((1,), (1,)), ((), ())),
                         preferred_element_type=_f32)
    out_ref[...] = xw + b_ref[...][0]


def _dense_out(xt, wout, bout):
    return pl.pallas_call(
        _dense_out_body,
        grid=(NBLK,),
        in_specs=[
            pl.BlockSpec((1, BN, HD), lambda i: (0, i, 0)),
            pl.BlockSpec((1, BN, HD), lambda i: (1, i, 0)),
            pl.BlockSpec((FIN, D), lambda i: (0, 0)),
            pl.BlockSpec((8, FIN), lambda i: (0, 0)),
        ],
        out_specs=pl.BlockSpec((BN, FIN), lambda i: (i, 0)),
        out_shape=jax.ShapeDtypeStruct((NPT, FIN), _f32),
    )(xt, xt, wout, bout)


def kernel(x, H, vertex, edges, degV, degE, W0, b0, Wc, Wout, bout):
    vertex = vertex.astype(_i32)
    edges = edges.astype(_i32)
    zpad = jnp.zeros((PADP,), _i32)
    vg = jnp.concatenate([vertex, zpad])
    vgi = jnp.concatenate([vg, vg + NPT]).reshape(NC * IROWS, CW)
    esi = jnp.concatenate([edges, jnp.full((PADP,), M, _i32)]).reshape(IROWS, CW)

    vm = jnp.concatenate([vertex, jnp.full((PADP,), jnp.int32(1 << 30))])
    ep = jnp.concatenate([edges, zpad])
    egi_h = []
    vsi_h = []
    for h in range(2):
        mask = (vm >= h * VH) & (vm < (h + 1) * VH)
        eg = jnp.where(mask, ep, ZROW)
        egi_h.append(jnp.concatenate([eg, eg + NPT]).reshape(NC * IROWS, CW))
        vsi_h.append(jnp.where(mask, vm - h * VH, 0).reshape(IROWS, CW))

    b0r = jnp.broadcast_to(b0.reshape(1, D), (8, D))
    boutr = jnp.broadcast_to(bout.reshape(1, FIN), (8, FIN))

    x0, xt = _dense_in(x, W0, b0r)
    xt0_flat = xt.reshape(TROWS, HD)

    gsel = jnp.array([0, 1, 2, 3, 1, 2, 3], _i32)
    ssel = jnp.array([0, 0, 1, 2, 0, 1, 2], _i32)
    brs = jnp.array([0, 1, 2, 3, 1, 2, 4], _i32)
    gidx_all = jnp.stack([jnp.zeros((NC * IROWS, CW), _i32), vgi,
                          egi_h[0], egi_h[1]])
    sidx_all = jnp.stack([esi, vsi_h[0], vsi_h[1]])
    ones_tab = jnp.ones((TROWS, HD), _f32)

    def loop_body(t, state):
        table, cnt, prev = state
        gidx = lax.dynamic_index_in_dim(gidx_all, gsel[t], 0, keepdims=False)
        sidx = lax.dynamic_index_in_dim(sidx_all, ssel[t], 0, keepdims=False)
        pa = _sc_pass(table, gidx, sidx)

        def b_counts(_):
            return xt0_flat, lax.slice(pa, (0, 0), (MC, 16)), pa

        def b_mk_xe(_):
            return _mk_xe(pa, cnt, degE), cnt, pa

        def b_stash(_):
            return table, cnt, pa

        def b_fin0(_):
            return (_layer_fin[0](prev, pa, degV, x0, Wc[0]).reshape(TROWS, HD),
                    cnt, pa)

        def b_fin1(_):
            return (_layer_fin[1](prev, pa, degV, x0, Wc[1]).reshape(TROWS, HD),
                    cnt, pa)

        return lax.switch(brs[t], [b_counts, b_mk_xe, b_stash, b_fin0, b_fin1],
                          0)

    cnt0 = jnp.zeros((MC, 16), _f32)
    prev0 = jnp.zeros((NC * AROWS, HD), _f32)
    table_fin, _, _ = lax.fori_loop(0, 7, loop_body, (ones_tab, cnt0, prev0))

    out = _dense_out(table_fin.reshape(NC, NPT, HD), Wout, boutr)
    return out[:N]


# speedup vs baseline: 1.8388x; 1.8220x over previous
import functools
import math

import jax
import jax.numpy as jnp
from jax import lax
from jax.experimental import pallas as pl
from jax.experimental.pallas import tpu as pltpu
from jax.experimental.pallas import tpu_sc as plsc

N = 50000
M = 25000
E = 800000
FIN = 128
D = 64
HD = 32

NC = 2
NS = 16

CW = 64
CH = 784
PAIRS_PAD = NS * CH * CW
PADP = PAIRS_PAD - E
IROWS = PAIRS_PAD // CW

NPT = 50176
TROWS = NC * NPT
AROWS = 26624
IB = 16
NBUF = 8
VH = AROWS
MC = 25088
ZBLK = 97
ZROW = ZBLK * 512

BN = 512
NBLK = NPT // BN
HBLK = AROWS // BN

_f32 = jnp.float32
_i32 = jnp.int32

PER_TILE = AROWS // NS


def _fill_vmem(ref, rows, cols, value):
    def body(r, carry):
        for j in range(cols // 16):
            ref[r, pl.ds(16 * j, 16)] = jnp.full((16,), value, _f32)
        return carry
    lax.fori_loop(0, rows, body, 0)


@functools.partial(
    pl.kernel,
    out_type=jax.ShapeDtypeStruct((NC * AROWS, HD), _f32),
    mesh=plsc.VectorSubcoreMesh(core_axis_name="c", subcore_axis_name="s"),
    scratch_types=[
        pltpu.VMEM((IB, CW), _i32),
        pltpu.VMEM((IB, CW), _i32),
    ] + [pltpu.VMEM((CW, HD), _f32) for _ in range(NBUF)] + [
        pltpu.VMEM_SHARED((AROWS, HD), _f32),
        pltpu.SemaphoreType.DMA,
        pltpu.SemaphoreType.DMA,
    ],
    compiler_params=pltpu.CompilerParams(use_tc_tiling_on_sc=False,
                                         has_side_effects=True),
)
def _sc_pass(table, gidx_hbm, sidx_hbm, out, gidx, sidx, *rest):
    rows = list(rest[:NBUF])
    acc, semg, sems = rest[NBUF], rest[NBUF + 1], rest[NBUF + 2]
    c = lax.axis_index("c")
    s = lax.axis_index("s")
    r0 = rows[0]
    _fill_vmem(r0, CW, HD, 0.0)
    def zacc(k, carry):
        pltpu.sync_copy(r0, acc.at[pl.ds(s * PER_TILE + k * CW, CW)])
        return carry
    lax.fori_loop(0, PER_TILE // CW, zacc, 0)
    plsc.subcore_barrier()

    def superchunk(u, carry):
        pltpu.sync_copy(gidx_hbm.at[pl.ds((c * NS + s) * CH + u * IB, IB)],
                        gidx)
        pltpu.sync_copy(sidx_hbm.at[pl.ds(s * CH + u * IB, IB)], sidx)
        for g in range(IB // NBUF):
            gets = [pltpu.async_copy(table.at[gidx.at[g * NBUF + b]], rows[b],
                                     semg) for b in range(NBUF)]
            for d in gets:
                d.wait()
            puts = [pltpu.async_copy(rows[b], acc.at[sidx.at[g * NBUF + b]],
                                     sems, add=True) for b in range(NBUF)]
            for d in puts:
                d.wait()
        return carry

    lax.fori_loop(0, CH // IB, superchunk, 0)
    plsc.subcore_barrier()
    pltpu.sync_copy(acc.at[pl.ds(s * PER_TILE, PER_TILE)],
                    out.at[pl.ds(c * AROWS + s * PER_TILE, PER_TILE)])


def _dense_in_body(x_ref, w_ref, b_ref, x0_ref, xt_ref):
    xb = x_ref[...]
    xw = lax.dot_general(xb, w_ref[...], (((1,), (1,)), ((), ())),
                         preferred_element_type=_f32)
    xw = jnp.maximum(xw + b_ref[...][0], 0.0)
    x0_ref[...] = xw
    xt_ref[0] = xw[:, :HD]
    xt_ref[1] = xw[:, HD:]


def _dense_in(x, w0, b0):
    return pl.pallas_call(
        _dense_in_body,
        grid=(NBLK,),
        in_specs=[
            pl.BlockSpec((BN, FIN), lambda i: (i, 0)),
            pl.BlockSpec((D, FIN), lambda i: (0, 0)),
            pl.BlockSpec((8, D), lambda i: (0, 0)),
        ],
        out_specs=[
            pl.BlockSpec((BN, D), lambda i: (i, 0)),
            pl.BlockSpec((2, BN, HD), lambda i: (0, i, 0)),
        ],
        out_shape=[
            jax.ShapeDtypeStruct((NPT, D), _f32),
            jax.ShapeDtypeStruct((2, NPT, HD), _f32),
        ],
    )(x, w0, b0)


def _mk_xe_body(pa_ref, cnt_ref, dege_ref, out_ref):
    j = pl.program_id(0) % 50

    @pl.when(j < 49)
    def _():
        cnt = cnt_ref[...][:, 0:1]
        scale = dege_ref[...] / jnp.maximum(cnt, 1.0)
        out_ref[...] = pa_ref[...] * scale

    @pl.when(j == 49)
    def _():
        out_ref[...] = jnp.zeros((BN, HD), _f32)


def _mk_xe(pa, cnt, dege):
    return pl.pallas_call(
        _mk_xe_body,
        grid=(NC * 50,),
        in_specs=[
            pl.BlockSpec(
                (BN, HD),
                lambda i: (HBLK * (i // 50) + jnp.minimum(i % 50, 48), 0)),
            pl.BlockSpec((BN, 16), lambda i: (jnp.minimum(i % 50, 48), 0)),
            pl.BlockSpec((BN, 1), lambda i: (jnp.minimum(i % 50, 48), 0)),
        ],
        out_specs=pl.BlockSpec(
            (BN, HD),
            lambda i: (NBLK * (i // 50) + jnp.where(i % 50 < 49, i % 50, ZBLK),
                       0)),
        out_shape=jax.ShapeDtypeStruct((TROWS, HD), _f32),
    )(pa, cnt, dege)


def _make_layer_fin(beta):
    def body(h0lo_ref, h0hi_ref, h1lo_ref, h1hi_ref, degv_ref, x0_ref, wc_ref,
             out_ref):
        use0 = pl.program_id(0) < HBLK
        xlo = jnp.where(use0, h0lo_ref[...], h1lo_ref[...])
        xhi = jnp.where(use0, h0hi_ref[...], h1hi_ref[...])
        xv = jnp.concatenate([xlo, xhi], axis=1)
        xv = xv * degv_ref[...]
        nrm = jnp.sqrt(jnp.sum(xv * xv, axis=1, keepdims=True))
        scale = jnp.where(nrm > 0, 1.0 / jnp.maximum(nrm, 1e-30), 0.0)
        xi = 0.9 * (xv * scale) + 0.1 * x0_ref[...]
        xw = lax.dot_general(xi, wc_ref[...], (((1,), (1,)), ((), ())),
                             preferred_element_type=_f32)
        xl = jnp.maximum((1.0 - beta) * xi + beta * xw, 0.0)
        out_ref[0] = xl[:, :HD]
        out_ref[1] = xl[:, HD:]

    def run(pa0, pa1, degv, x0, wc):
        h1b = lambda i: jnp.clip(i - HBLK, 0, 45)
        return pl.pallas_call(
            body,
            grid=(NBLK,),
            in_specs=[
                pl.BlockSpec((BN, HD), lambda i: (jnp.minimum(i, HBLK - 1), 0)),
                pl.BlockSpec((BN, HD),
                             lambda i: (HBLK + jnp.minimum(i, HBLK - 1), 0)),
                pl.BlockSpec((BN, HD), lambda i: (h1b(i), 0)),
                pl.BlockSpec((BN, HD), lambda i: (HBLK + h1b(i), 0)),
                pl.BlockSpec((BN, 1), lambda i: (i, 0)),
                pl.BlockSpec((BN, D), lambda i: (i, 0)),
                pl.BlockSpec((D, D), lambda i: (0, 0)),
            ],
            out_specs=pl.BlockSpec((2, BN, HD), lambda i: (0, i, 0)),
            out_shape=jax.ShapeDtypeStruct((2, NPT, HD), _f32),
        )(pa0, pa0, pa1, pa1, degv, x0, wc)

    return run


_layer_fin = [_make_layer_fin(math.log(0.5 / (i + 1) + 1.0)) for i in range(2)]


def _dense_out_body(lo_ref, hi_ref, w_ref, b_ref, out_ref):
    xb = jnp.concatenate([lo_ref[0], hi_ref[0]], axis=1)
    xw = lax.dot_general(xb, w_ref[...], (((1,), (1,)), ((), ())),
                         preferred_element_type=_f32)
    out_ref[...] = xw + b_ref[...][0]


def _dense_out(xt, wout, bout):
    return pl.pallas_call(
        _dense_out_body,
        grid=(NBLK,),
        in_specs=[
            pl.BlockSpec((1, BN, HD), lambda i: (0, i, 0)),
            pl.BlockSpec((1, BN, HD), lambda i: (1, i, 0)),
            pl.BlockSpec((FIN, D), lambda i: (0, 0)),
            pl.BlockSpec((8, FIN), lambda i: (0, 0)),
        ],
        out_specs=pl.BlockSpec((BN, FIN), lambda i: (i, 0)),
        out_shape=jax.ShapeDtypeStruct((NPT, FIN), _f32),
    )(xt, xt, wout, bout)


def kernel(x, H, vertex, edges, degV, degE, W0, b0, Wc, Wout, bout):
    vertex = vertex.astype(_i32)
    edges = edges.astype(_i32)
    zpad = jnp.zeros((PADP,), _i32)
    vg = jnp.concatenate([vertex, zpad])
    vgi = jnp.concatenate([vg, vg + NPT]).reshape(NC * IROWS, CW)
    esi = jnp.concatenate([edges, jnp.full((PADP,), M, _i32)]).reshape(IROWS, CW)

    vm = jnp.concatenate([vertex, jnp.full((PADP,), jnp.int32(1 << 30))])
    ep = jnp.concatenate([edges, zpad])
    egi_h = []
    vsi_h = []
    for h in range(2):
        mask = (vm >= h * VH) & (vm < (h + 1) * VH)
        eg = jnp.where(mask, ep, ZROW)
        egi_h.append(jnp.concatenate([eg, eg + NPT]).reshape(NC * IROWS, CW))
        vsi_h.append(jnp.where(mask, vm - h * VH, 0).reshape(IROWS, CW))

    b0r = jnp.broadcast_to(b0.reshape(1, D), (8, D))
    boutr = jnp.broadcast_to(bout.reshape(1, FIN), (8, FIN))

    x0, xt = _dense_in(x, W0, b0r)
    xt0_flat = xt.reshape(TROWS, HD)

    gsel = jnp.array([0, 1, 2, 3, 1, 2, 3], _i32)
    ssel = jnp.array([0, 0, 1, 2, 0, 1, 2], _i32)
    brs = jnp.array([0, 1, 2, 3, 1, 2, 4], _i32)
    spread = (jnp.arange(NC * PAIRS_PAD, dtype=_i32) % TROWS).reshape(
        NC * IROWS, CW)
    gidx_all = jnp.stack([spread, vgi, egi_h[0], egi_h[1]])
    sidx_all = jnp.stack([esi, vsi_h[0], vsi_h[1]])
    ones_tab = jnp.ones((TROWS, HD), _f32)

    def loop_body(t, state):
        table, cnt, prev = state
        gidx = lax.dynamic_index_in_dim(gidx_all, gsel[t], 0, keepdims=False)
        sidx = lax.dynamic_index_in_dim(sidx_all, ssel[t], 0, keepdims=False)
        pa = _sc_pass(table, gidx, sidx)

        def b_counts(_):
            return xt0_flat, lax.slice(pa, (0, 0), (MC, 16)), pa

        def b_mk_xe(_):
            return _mk_xe(pa, cnt, degE), cnt, pa

        def b_stash(_):
            return table, cnt, pa

        def b_fin0(_):
            return (_layer_fin[0](prev, pa, degV, x0, Wc[0]).reshape(TROWS, HD),
                    cnt, pa)

        def b_fin1(_):
            return (_layer_fin[1](prev, pa, degV, x0, Wc[1]).reshape(TROWS, HD),
                    cnt, pa)

        return lax.switch(brs[t], [b_counts, b_mk_xe, b_stash, b_fin0, b_fin1],
                          0)

    cnt0 = jnp.zeros((MC, 16), _f32)
    prev0 = jnp.zeros((NC * AROWS, HD), _f32)
    table_fin, _, _ = lax.fori_loop(0, 7, loop_body, (ones_tab, cnt0, prev0))

    out = _dense_out(table_fin.reshape(NC, NPT, HD), Wout, boutr)
    return out[:N]


# speedup vs baseline: 7.5579x; 4.1102x over previous
import functools
import math

import jax
import jax.numpy as jnp
from jax import lax
from jax.experimental import pallas as pl
from jax.experimental.pallas import tpu as pltpu
from jax.experimental.pallas import tpu_sc as plsc

N = 50000
M = 25000
E = 800000
FIN = 128
D = 64
HD = 32

NC = 2
NS = 16

CW = 64
CH = 784
PAIRS_PAD = NS * CH * CW
PADP = PAIRS_PAD - E
IROWS = PAIRS_PAD // CW

NPT = 50176
TROWS = NC * NPT
AROWS = 26624
IB = 16
NBUF = 8
VH = AROWS
MC = 25088
ZBLK = 97
ZROW = ZBLK * 512

BN = 512
NBLK = NPT // BN
HBLK = AROWS // BN

_f32 = jnp.float32
_i32 = jnp.int32

PER_TILE = AROWS // NS


def _fill_vmem(ref, rows, cols, value):
    def body(r, carry):
        for j in range(cols // 16):
            ref[r, pl.ds(16 * j, 16)] = jnp.full((16,), value, _f32)
        return carry
    lax.fori_loop(0, rows, body, 0)


@functools.partial(
    pl.kernel,
    out_type=jax.ShapeDtypeStruct((NC * AROWS, HD), _f32),
    mesh=plsc.VectorSubcoreMesh(core_axis_name="c", subcore_axis_name="s"),
    scratch_types=[
        pltpu.VMEM((IB, CW), _i32),
        pltpu.VMEM((IB, CW), _i32),
    ] + [pltpu.VMEM((CW, HD), _f32) for _ in range(NBUF)] + [
        pltpu.VMEM_SHARED((AROWS, HD), _f32),
        pltpu.SemaphoreType.DMA,
        pltpu.SemaphoreType.DMA,
    ],
    compiler_params=pltpu.CompilerParams(use_tc_tiling_on_sc=False,
                                         has_side_effects=True),
)
def _sc_pass(table, gidx_hbm, sidx_hbm, out, gidx, sidx, *rest):
    rows = list(rest[:NBUF])
    acc, semg, sems = rest[NBUF], rest[NBUF + 1], rest[NBUF + 2]
    c = lax.axis_index("c")
    s = lax.axis_index("s")
    r0 = rows[0]
    _fill_vmem(r0, CW, HD, 0.0)
    def zacc(k, carry):
        pltpu.sync_copy(r0, acc.at[pl.ds(s * PER_TILE + k * CW, CW)])
        return carry
    lax.fori_loop(0, PER_TILE // CW, zacc, 0)
    plsc.subcore_barrier()

    def superchunk(u, carry):
        pltpu.sync_copy(gidx_hbm.at[pl.ds((c * NS + s) * CH + u * IB, IB)],
                        gidx)
        pltpu.sync_copy(sidx_hbm.at[pl.ds(s * CH + u * IB, IB)], sidx)
        for g in range(IB // NBUF):
            gets = [pltpu.async_copy(table.at[gidx.at[g * NBUF + b]], rows[b],
                                     semg) for b in range(NBUF)]
            for d in gets:
                d.wait()
            puts = [pltpu.async_copy(rows[b], acc.at[sidx.at[g * NBUF + b]],
                                     sems, add=True) for b in range(NBUF)]
            for d in puts:
                d.wait()
        return carry

    lax.fori_loop(0, CH // IB, superchunk, 0)
    plsc.subcore_barrier()
    pltpu.sync_copy(acc.at[pl.ds(s * PER_TILE, PER_TILE)],
                    out.at[pl.ds(c * AROWS + s * PER_TILE, PER_TILE)])


def _dense_in_body(x_ref, w_ref, b_ref, x0_ref, xt_ref):
    xb = x_ref[...]
    xw = lax.dot_general(xb, w_ref[...], (((1,), (1,)), ((), ())),
                         preferred_element_type=_f32)
    xw = jnp.maximum(xw + b_ref[...][0], 0.0)
    x0_ref[...] = xw
    xt_ref[0] = xw[:, :HD]
    xt_ref[1] = xw[:, HD:]


def _dense_in(x, w0, b0):
    return pl.pallas_call(
        _dense_in_body,
        grid=(NBLK,),
        in_specs=[
            pl.BlockSpec((BN, FIN), lambda i: (i, 0)),
            pl.BlockSpec((D, FIN), lambda i: (0, 0)),
            pl.BlockSpec((8, D), lambda i: (0, 0)),
        ],
        out_specs=[
            pl.BlockSpec((BN, D), lambda i: (i, 0)),
            pl.BlockSpec((2, BN, HD), lambda i: (0, i, 0)),
        ],
        out_shape=[
            jax.ShapeDtypeStruct((NPT, D), _f32),
            jax.ShapeDtypeStruct((2, NPT, HD), _f32),
        ],
    )(x, w0, b0)


def _mk_xe_body(pa_ref, cnt_ref, dege_ref, out_ref):
    j = pl.program_id(0) % 50

    @pl.when(j < 49)
    def _():
        cnt = cnt_ref[...][:, 0:1]
        scale = dege_ref[...] / jnp.maximum(cnt, 1.0)
        out_ref[...] = pa_ref[...] * scale

    @pl.when(j == 49)
    def _():
        out_ref[...] = jnp.zeros((BN, HD), _f32)


def _mk_xe(pa, cnt, dege):
    return pl.pallas_call(
        _mk_xe_body,
        grid=(NC * 50,),
        in_specs=[
            pl.BlockSpec(
                (BN, HD),
                lambda i: (HBLK * (i // 50) + jnp.minimum(i % 50, 48), 0)),
            pl.BlockSpec((BN, 16), lambda i: (jnp.minimum(i % 50, 48), 0)),
            pl.BlockSpec((BN, 1), lambda i: (jnp.minimum(i % 50, 48), 0)),
        ],
        out_specs=pl.BlockSpec(
            (BN, HD),
            lambda i: (NBLK * (i // 50) + jnp.where(i % 50 < 49, i % 50, ZBLK),
                       0)),
        out_shape=jax.ShapeDtypeStruct((TROWS, HD), _f32),
    )(pa, cnt, dege)


def _make_layer_fin(beta):
    def body(h0lo_ref, h0hi_ref, h1lo_ref, h1hi_ref, degv_ref, x0_ref, wc_ref,
             out_ref):
        use0 = pl.program_id(0) < HBLK
        xlo = jnp.where(use0, h0lo_ref[...], h1lo_ref[...])
        xhi = jnp.where(use0, h0hi_ref[...], h1hi_ref[...])
        xv = jnp.concatenate([xlo, xhi], axis=1)
        xv = xv * degv_ref[...]
        nrm = jnp.sqrt(jnp.sum(xv * xv, axis=1, keepdims=True))
        scale = jnp.where(nrm > 0, 1.0 / jnp.maximum(nrm, 1e-30), 0.0)
        xi = 0.9 * (xv * scale) + 0.1 * x0_ref[...]
        xw = lax.dot_general(xi, wc_ref[...], (((1,), (1,)), ((), ())),
                             preferred_element_type=_f32)
        xl = jnp.maximum((1.0 - beta) * xi + beta * xw, 0.0)
        out_ref[0] = xl[:, :HD]
        out_ref[1] = xl[:, HD:]

    def run(pa0, pa1, degv, x0, wc):
        h1b = lambda i: jnp.clip(i - HBLK, 0, 45)
        return pl.pallas_call(
            body,
            grid=(NBLK,),
            in_specs=[
                pl.BlockSpec((BN, HD), lambda i: (jnp.minimum(i, HBLK - 1), 0)),
                pl.BlockSpec((BN, HD),
                             lambda i: (HBLK + jnp.minimum(i, HBLK - 1), 0)),
                pl.BlockSpec((BN, HD), lambda i: (h1b(i), 0)),
                pl.BlockSpec((BN, HD), lambda i: (HBLK + h1b(i), 0)),
                pl.BlockSpec((BN, 1), lambda i: (i, 0)),
                pl.BlockSpec((BN, D), lambda i: (i, 0)),
                pl.BlockSpec((D, D), lambda i: (0, 0)),
            ],
            out_specs=pl.BlockSpec((2, BN, HD), lambda i: (0, i, 0)),
            out_shape=jax.ShapeDtypeStruct((2, NPT, HD), _f32),
        )(pa0, pa0, pa1, pa1, degv, x0, wc)

    return run


_layer_fin = [_make_layer_fin(math.log(0.5 / (i + 1) + 1.0)) for i in range(2)]


def _dense_out_body(lo_ref, hi_ref, w_ref, b_ref, out_ref):
    xb = jnp.concatenate([lo_ref[0], hi_ref[0]], axis=1)
    xw = lax.dot_general(xb, w_ref[...], (((1,), (1,)), ((), ())),
                         preferred_element_type=_f32)
    out_ref[...] = xw + b_ref[...][0]


def _dense_out(xt, wout, bout):
    return pl.pallas_call(
        _dense_out_body,
        grid=(NBLK,),
        in_specs=[
            pl.BlockSpec((1, BN, HD), lambda i: (0, i, 0)),
            pl.BlockSpec((1, BN, HD), lambda i: (1, i, 0)),
            pl.BlockSpec((FIN, D), lambda i: (0, 0)),
            pl.BlockSpec((8, FIN), lambda i: (0, 0)),
        ],
        out_specs=pl.BlockSpec((BN, FIN), lambda i: (i, 0)),
        out_shape=jax.ShapeDtypeStruct((NPT, FIN), _f32),
    )(xt, xt, wout, bout)


def kernel(x, H, vertex, edges, degV, degE, W0, b0, Wc, Wout, bout):
    vertex = vertex.astype(_i32)
    edges = edges.astype(_i32)
    zpad = jnp.zeros((PADP,), _i32)
    vg = jnp.concatenate([vertex, zpad])
    vgi = jnp.concatenate([vg, vg + NPT]).reshape(NC * IROWS, CW)
    esi = jnp.concatenate([edges, jnp.full((PADP,), M, _i32)]).reshape(IROWS, CW)

    vm = jnp.concatenate([vertex, jnp.full((PADP,), jnp.int32(1 << 30))])
    ep = jnp.concatenate([edges, zpad])
    zspread = ZROW + (jnp.arange(PAIRS_PAD, dtype=_i32) % 512)
    egi_h = []
    vsi_h = []
    for h in range(2):
        mask = (vm >= h * VH) & (vm < (h + 1) * VH)
        eg = jnp.where(mask, ep, zspread)
        egi_h.append(jnp.concatenate([eg, eg + NPT]).reshape(NC * IROWS, CW))
        vsi_h.append(jnp.where(mask, vm - h * VH, 0).reshape(IROWS, CW))

    b0r = jnp.broadcast_to(b0.reshape(1, D), (8, D))
    boutr = jnp.broadcast_to(bout.reshape(1, FIN), (8, FIN))

    x0, xt = _dense_in(x, W0, b0r)
    xt0_flat = xt.reshape(TROWS, HD)

    gsel = jnp.array([0, 1, 2, 3, 1, 2, 3], _i32)
    ssel = jnp.array([0, 0, 1, 2, 0, 1, 2], _i32)
    brs = jnp.array([0, 1, 2, 3, 1, 2, 4], _i32)
    spread = (jnp.arange(NC * PAIRS_PAD, dtype=_i32) % TROWS).reshape(
        NC * IROWS, CW)
    gidx_all = jnp.stack([spread, vgi, egi_h[0], egi_h[1]])
    sidx_all = jnp.stack([esi, vsi_h[0], vsi_h[1]])
    ones_tab = jnp.ones((TROWS, HD), _f32)

    def loop_body(t, state):
        table, cnt, prev = state
        gidx = lax.dynamic_index_in_dim(gidx_all, gsel[t], 0, keepdims=False)
        sidx = lax.dynamic_index_in_dim(sidx_all, ssel[t], 0, keepdims=False)
        pa = _sc_pass(table, gidx, sidx)

        def b_counts(_):
            return xt0_flat, lax.slice(pa, (0, 0), (MC, 16)), pa

        def b_mk_xe(_):
            return _mk_xe(pa, cnt, degE), cnt, pa

        def b_stash(_):
            return table, cnt, pa

        def b_fin0(_):
            return (_layer_fin[0](prev, pa, degV, x0, Wc[0]).reshape(TROWS, HD),
                    cnt, pa)

        def b_fin1(_):
            return (_layer_fin[1](prev, pa, degV, x0, Wc[1]).reshape(TROWS, HD),
                    cnt, pa)

        return lax.switch(brs[t], [b_counts, b_mk_xe, b_stash, b_fin0, b_fin1],
                          0)

    cnt0 = jnp.zeros((MC, 16), _f32)
    prev0 = jnp.zeros((NC * AROWS, HD), _f32)
    table_fin, _, _ = lax.fori_loop(0, 7, loop_body, (ones_tab, cnt0, prev0))

    out = _dense_out(table_fin.reshape(NC, NPT, HD), Wout, boutr)
    return out[:N]


# speedup vs baseline: 12.9280x; 1.7105x over previous
import functools
import math

import jax
import jax.numpy as jnp
from jax import lax
from jax.experimental import pallas as pl
from jax.experimental.pallas import tpu as pltpu
from jax.experimental.pallas import tpu_sc as plsc

N = 50000
M = 25000
E = 800000
FIN = 128
D = 64
HD = 32

NC = 2
NS = 16

CW = 64
CH = 784
PAIRS_PAD = NS * CH * CW
PADP = PAIRS_PAD - E
IROWS = PAIRS_PAD // CW
IB = 16
NBUF = 8

NPT = 50176
TROWS = NC * NPT
AROWS = 51200
MC = 25088

BN = 512
NBLK = NPT // BN
ABLK = AROWS // BN

_f32 = jnp.float32
_bf16 = jnp.bfloat16
_i32 = jnp.int32

PER_TILE = AROWS // NS


def _zero_vmem_bf16(ref, rows):
    def body(r, carry):
        ref[r, pl.ds(0, 32)] = jnp.zeros((32,), _bf16)
        return carry
    lax.fori_loop(0, rows, body, 0)


@functools.partial(
    pl.kernel,
    out_type=jax.ShapeDtypeStruct((NC * AROWS, HD), _bf16),
    mesh=plsc.VectorSubcoreMesh(core_axis_name="c", subcore_axis_name="s"),
    scratch_types=[
        pltpu.VMEM((IB, CW), _i32),
        pltpu.VMEM((IB, CW), _i32),
    ] + [pltpu.VMEM((CW, HD), _bf16) for _ in range(NBUF)] + [
        pltpu.VMEM_SHARED((AROWS, HD), _bf16),
        pltpu.SemaphoreType.DMA,
        pltpu.SemaphoreType.DMA,
    ],
    compiler_params=pltpu.CompilerParams(use_tc_tiling_on_sc=False,
                                         has_side_effects=True),
)
def _sc_pass(table, gidx_hbm, sidx_hbm, out, gidx, sidx, *rest):
    rows = list(rest[:NBUF])
    acc, semg, sems = rest[NBUF], rest[NBUF + 1], rest[NBUF + 2]
    c = lax.axis_index("c")
    s = lax.axis_index("s")
    r0 = rows[0]
    _zero_vmem_bf16(r0, CW)
    def zacc(k, carry):
        pltpu.sync_copy(r0, acc.at[pl.ds(s * PER_TILE + k * CW, CW)])
        return carry
    lax.fori_loop(0, PER_TILE // CW, zacc, 0)
    plsc.subcore_barrier()

    def superchunk(u, carry):
        pltpu.sync_copy(gidx_hbm.at[pl.ds((c * NS + s) * CH + u * IB, IB)],
                        gidx)
        pltpu.sync_copy(sidx_hbm.at[pl.ds(s * CH + u * IB, IB)], sidx)
        for g in range(IB // NBUF):
            gets = [pltpu.async_copy(table.at[gidx.at[g * NBUF + b]], rows[b],
                                     semg) for b in range(NBUF)]
            for d in gets:
                d.wait()
            puts = [pltpu.async_copy(rows[b], acc.at[sidx.at[g * NBUF + b]],
                                     sems, add=True) for b in range(NBUF)]
            for d in puts:
                d.wait()
        return carry

    lax.fori_loop(0, CH // IB, superchunk, 0)
    plsc.subcore_barrier()
    pltpu.sync_copy(acc.at[pl.ds(s * PER_TILE, PER_TILE)],
                    out.at[pl.ds(c * AROWS + s * PER_TILE, PER_TILE)])


def _dense_in_body(x_ref, w_ref, b_ref, x0_ref, xt_ref):
    xb = x_ref[...]
    xw = lax.dot_general(xb, w_ref[...], (((1,), (1,)), ((), ())),
                         preferred_element_type=_f32)
    xw = jnp.maximum(xw + b_ref[...][0], 0.0)
    x0_ref[...] = xw
    xt_ref[0] = xw[:, :HD].astype(_bf16)
    xt_ref[1] = xw[:, HD:].astype(_bf16)


def _dense_in(x, w0, b0):
    return pl.pallas_call(
        _dense_in_body,
        grid=(NBLK,),
        in_specs=[
            pl.BlockSpec((BN, FIN), lambda i: (i, 0)),
            pl.BlockSpec((D, FIN), lambda i: (0, 0)),
            pl.BlockSpec((8, D), lambda i: (0, 0)),
        ],
        out_specs=[
            pl.BlockSpec((BN, D), lambda i: (i, 0)),
            pl.BlockSpec((2, BN, HD), lambda i: (0, i, 0)),
        ],
        out_shape=[
            jax.ShapeDtypeStruct((NPT, D), _f32),
            jax.ShapeDtypeStruct((2, NPT, HD), _bf16),
        ],
    )(x, w0, b0)


def _mk_xe_body(pa_ref, cnt_ref, dege_ref, out_ref):
    cnt = cnt_ref[...][:, 0:1].astype(_f32)
    scale = dege_ref[...] / jnp.maximum(cnt, 1.0)
    out_ref[...] = (pa_ref[...].astype(_f32) * scale).astype(_bf16)


def _mk_xe(pa, cnt, dege):
    return pl.pallas_call(
        _mk_xe_body,
        grid=(NC * 49,),
        in_specs=[
            pl.BlockSpec((BN, HD), lambda i: (ABLK * (i // 49) + i % 49, 0)),
            pl.BlockSpec((BN, 16), lambda i: (i % 49, 0)),
            pl.BlockSpec((BN, 1), lambda i: (i % 49, 0)),
        ],
        out_specs=pl.BlockSpec(
            (BN, HD), lambda i: (NBLK * (i // 49) + i % 49, 0)),
        out_shape=jax.ShapeDtypeStruct((TROWS, HD), _bf16),
    )(pa, cnt, dege)


def _make_layer_fin(beta):
    def body(lo_ref, hi_ref, degv_ref, x0_ref, wc_ref, bf_ref, f32_ref):
        xv = jnp.concatenate([lo_ref[...].astype(_f32),
                              hi_ref[...].astype(_f32)], axis=1)
        xv = xv * degv_ref[...]
        nrm = jnp.sqrt(jnp.sum(xv * xv, axis=1, keepdims=True))
        scale = jnp.where(nrm > 0, 1.0 / jnp.maximum(nrm, 1e-30), 0.0)
        xi = 0.9 * (xv * scale) + 0.1 * x0_ref[...]
        xw = lax.dot_general(xi, wc_ref[...], (((1,), (1,)), ((), ())),
                             preferred_element_type=_f32)
        xl = jnp.maximum((1.0 - beta) * xi + beta * xw, 0.0)
        bf_ref[0] = xl[:, :HD].astype(_bf16)
        bf_ref[1] = xl[:, HD:].astype(_bf16)
        f32_ref[0] = xl[:, :HD]
        f32_ref[1] = xl[:, HD:]

    def run(pa, degv, x0, wc):
        return pl.pallas_call(
            body,
            grid=(NBLK,),
            in_specs=[
                pl.BlockSpec((BN, HD), lambda i: (i, 0)),
                pl.BlockSpec((BN, HD), lambda i: (ABLK + i, 0)),
                pl.BlockSpec((BN, 1), lambda i: (i, 0)),
                pl.BlockSpec((BN, D), lambda i: (i, 0)),
                pl.BlockSpec((D, D), lambda i: (0, 0)),
            ],
            out_specs=[
                pl.BlockSpec((2, BN, HD), lambda i: (0, i, 0)),
                pl.BlockSpec((2, BN, HD), lambda i: (0, i, 0)),
            ],
            out_shape=[
                jax.ShapeDtypeStruct((2, NPT, HD), _bf16),
                jax.ShapeDtypeStruct((2, NPT, HD), _f32),
            ],
        )(pa, pa, degv, x0, wc)

    return run


_layer_fin = [_make_layer_fin(math.log(0.5 / (i + 1) + 1.0)) for i in range(2)]


def _dense_out_body(lo_ref, hi_ref, w_ref, b_ref, out_ref):
    xb = jnp.concatenate([lo_ref[0], hi_ref[0]], axis=1)
    xw = lax.dot_general(xb, w_ref[...], (((1,), (1,)), ((), ())),
                         preferred_element_type=_f32)
    out_ref[...] = xw + b_ref[...][0]


def _dense_out(xt, wout, bout):
    return pl.pallas_call(
        _dense_out_body,
        grid=(NBLK,),
        in_specs=[
            pl.BlockSpec((1, BN, HD), lambda i: (0, i, 0)),
            pl.BlockSpec((1, BN, HD), lambda i: (1, i, 0)),
            pl.BlockSpec((FIN, D), lambda i: (0, 0)),
            pl.BlockSpec((8, FIN), lambda i: (0, 0)),
        ],
        out_specs=pl.BlockSpec((BN, FIN), lambda i: (i, 0)),
        out_shape=jax.ShapeDtypeStruct((NPT, FIN), _f32),
    )(xt, xt, wout, bout)


def kernel(x, H, vertex, edges, degV, degE, W0, b0, Wc, Wout, bout):
    vertex = vertex.astype(_i32)
    edges = edges.astype(_i32)
    zpad = jnp.zeros((PADP,), _i32)
    vg = jnp.concatenate([vertex, zpad])
    vgi = jnp.concatenate([vg, vg + NPT]).reshape(NC * IROWS, CW)
    esi = jnp.concatenate([edges, jnp.full((PADP,), M, _i32)]).reshape(IROWS, CW)
    eg = jnp.concatenate([edges, zpad])
    egi = jnp.concatenate([eg, eg + NPT]).reshape(NC * IROWS, CW)
    vsi = jnp.concatenate([vertex, jnp.full((PADP,), N, _i32)]).reshape(IROWS, CW)

    b0r = jnp.broadcast_to(b0.reshape(1, D), (8, D))
    boutr = jnp.broadcast_to(bout.reshape(1, FIN), (8, FIN))

    x0, xt = _dense_in(x, W0, b0r)
    xt0_flat = xt.reshape(TROWS, HD)

    gsel = jnp.array([0, 1, 2, 1, 2], _i32)
    ssel = jnp.array([0, 0, 1, 0, 1], _i32)
    brs = jnp.array([0, 1, 2, 1, 3], _i32)
    spread = (jnp.arange(NC * PAIRS_PAD, dtype=_i32) % TROWS).reshape(
        NC * IROWS, CW)
    gidx_all = jnp.stack([spread, vgi, egi])
    sidx_all = jnp.stack([esi, vsi])
    ones_tab = jnp.ones((TROWS, HD), _bf16)

    def loop_body(t, state):
        table, xf, cnt = state
        gidx = lax.dynamic_index_in_dim(gidx_all, gsel[t], 0, keepdims=False)
        sidx = lax.dynamic_index_in_dim(sidx_all, ssel[t], 0, keepdims=False)
        pa = _sc_pass(table, gidx, sidx)

        def b_counts(_):
            return xt0_flat, xf, lax.slice(pa, (0, 0), (MC, 16))

        def b_mk_xe(_):
            return _mk_xe(pa, cnt, degE), xf, cnt

        def b_fin0(_):
            bf, f32 = _layer_fin[0](pa, degV, x0, Wc[0])
            return bf.reshape(TROWS, HD), f32, cnt

        def b_fin1(_):
            bf, f32 = _layer_fin[1](pa, degV, x0, Wc[1])
            return bf.reshape(TROWS, HD), f32, cnt

        return lax.switch(brs[t], [b_counts, b_mk_xe, b_fin0, b_fin1], 0)

    cnt0 = jnp.zeros((MC, 16), _bf16)
    xf0 = jnp.zeros((2, NPT, HD), _f32)
    _, xf_fin, _ = lax.fori_loop(0, 5, loop_body, (ones_tab, xf0, cnt0))

    out = _dense_out(xf_fin, Wout, boutr)
    return out[:N]


# speedup vs baseline: 15.8332x; 1.2247x over previous
import functools
import math

import jax
import jax.numpy as jnp
from jax import lax
from jax.experimental import pallas as pl
from jax.experimental.pallas import tpu as pltpu
from jax.experimental.pallas import tpu_sc as plsc

N = 50000
M = 25000
E = 800000
FIN = 128
D = 64
HD = 32

NC = 2
NS = 16

CW = 64
CH = 784
PAIRS_PAD = NS * CH * CW
PADP = PAIRS_PAD - E
IROWS = PAIRS_PAD // CW
IB = 56
NBUF = 8

NPT = 50176
TROWS = NC * NPT
AROWS = 51200
MC = 25088

BN = 512
NBLK = NPT // BN
ABLK = AROWS // BN

_f32 = jnp.float32
_bf16 = jnp.bfloat16
_i32 = jnp.int32

PER_TILE = AROWS // NS


def _zero_vmem_bf16(ref, rows):
    def body(r, carry):
        ref[r, pl.ds(0, 32)] = jnp.zeros((32,), _bf16)
        return carry
    lax.fori_loop(0, rows, body, 0)


@functools.partial(
    pl.kernel,
    out_type=jax.ShapeDtypeStruct((NC * AROWS, HD), _bf16),
    mesh=plsc.VectorSubcoreMesh(core_axis_name="c", subcore_axis_name="s"),
    scratch_types=[
        pltpu.VMEM((IB, CW), _i32),
        pltpu.VMEM((IB, CW), _i32),
    ] + [pltpu.VMEM((CW, HD), _bf16) for _ in range(2 * NBUF)] + [
        pltpu.VMEM_SHARED((AROWS, HD), _bf16),
        pltpu.SemaphoreType.DMA,
        pltpu.SemaphoreType.DMA,
        pltpu.SemaphoreType.DMA,
        pltpu.SemaphoreType.DMA,
    ],
    compiler_params=pltpu.CompilerParams(use_tc_tiling_on_sc=False,
                                         has_side_effects=True),
)
def _sc_pass(table, gidx_hbm, sidx_hbm, out, gidx, sidx, *rest):
    bufs = [list(rest[:NBUF]), list(rest[NBUF:2 * NBUF])]
    acc = rest[2 * NBUF]
    semg = [rest[2 * NBUF + 1], rest[2 * NBUF + 2]]
    sems = [rest[2 * NBUF + 3], rest[2 * NBUF + 4]]
    c = lax.axis_index("c")
    s = lax.axis_index("s")
    r0 = bufs[0][0]
    _zero_vmem_bf16(r0, CW)
    zs = [pltpu.async_copy(r0, acc.at[pl.ds(s * PER_TILE + k * CW, CW)],
                           sems[0]) for k in range(PER_TILE // CW)]
    for d in zs:
        d.wait()
    plsc.subcore_barrier()

    NG = IB // NBUF

    def superchunk(u, carry):
        pltpu.sync_copy(gidx_hbm.at[pl.ds((c * NS + s) * CH + u * IB, IB)],
                        gidx)
        pltpu.sync_copy(sidx_hbm.at[pl.ds(s * CH + u * IB, IB)], sidx)

        def fire_g(g, st):
            return [pltpu.async_copy(table.at[gidx.at[g * NBUF + b]],
                                     bufs[st][b], semg[st])
                    for b in range(NBUF)]

        def fire_s(g, st):
            return [pltpu.async_copy(bufs[st][b], acc.at[sidx.at[g * NBUF + b]],
                                     sems[st], add=True)
                    for b in range(NBUF)]

        pend_g = [None, None]
        pend_s = [None, None]
        pend_g[0] = fire_g(0, 0)
        for g in range(NG):
            st = g % 2
            ot = 1 - st
            if g + 1 < NG:
                if pend_s[ot] is not None:
                    for d in pend_s[ot]:
                        d.wait()
                pend_g[ot] = fire_g(g + 1, ot)
            for d in pend_g[st]:
                d.wait()
            pend_s[st] = fire_s(g, st)
        for ps in pend_s:
            if ps is not None:
                for d in ps:
                    d.wait()
        return carry

    lax.fori_loop(0, CH // IB, superchunk, 0)
    plsc.subcore_barrier()
    pltpu.sync_copy(acc.at[pl.ds(s * PER_TILE, PER_TILE)],
                    out.at[pl.ds(c * AROWS + s * PER_TILE, PER_TILE)])


def _dense_in_body(x_ref, w_ref, b_ref, x0_ref, xt_ref):
    xb = x_ref[...]
    xw = lax.dot_general(xb, w_ref[...], (((1,), (1,)), ((), ())),
                         preferred_element_type=_f32)
    xw = jnp.maximum(xw + b_ref[...][0], 0.0)
    x0_ref[...] = xw
    xt_ref[0] = xw[:, :HD].astype(_bf16)
    xt_ref[1] = xw[:, HD:].astype(_bf16)


def _dense_in(x, w0, b0):
    return pl.pallas_call(
        _dense_in_body,
        grid=(NBLK,),
        in_specs=[
            pl.BlockSpec((BN, FIN), lambda i: (i, 0)),
            pl.BlockSpec((D, FIN), lambda i: (0, 0)),
            pl.BlockSpec((8, D), lambda i: (0, 0)),
        ],
        out_specs=[
            pl.BlockSpec((BN, D), lambda i: (i, 0)),
            pl.BlockSpec((2, BN, HD), lambda i: (0, i, 0)),
        ],
        out_shape=[
            jax.ShapeDtypeStruct((NPT, D), _f32),
            jax.ShapeDtypeStruct((2, NPT, HD), _bf16),
        ],
    )(x, w0, b0)


def _mk_xe_body(pa_ref, cnt_ref, dege_ref, out_ref):
    cnt = cnt_ref[...][:, 0:1].astype(_f32)
    scale = dege_ref[...] / jnp.maximum(cnt, 1.0)
    out_ref[...] = (pa_ref[...].astype(_f32) * scale).astype(_bf16)


def _mk_xe(pa, cnt, dege):
    return pl.pallas_call(
        _mk_xe_body,
        grid=(NC * 49,),
        in_specs=[
            pl.BlockSpec((BN, HD), lambda i: (ABLK * (i // 49) + i % 49, 0)),
            pl.BlockSpec((BN, 16), lambda i: (i % 49, 0)),
            pl.BlockSpec((BN, 1), lambda i: (i % 49, 0)),
        ],
        out_specs=pl.BlockSpec(
            (BN, HD), lambda i: (NBLK * (i // 49) + i % 49, 0)),
        out_shape=jax.ShapeDtypeStruct((TROWS, HD), _bf16),
    )(pa, cnt, dege)


def _make_layer_fin(beta):
    def body(lo_ref, hi_ref, degv_ref, x0_ref, wc_ref, bf_ref, f32_ref):
        xv = jnp.concatenate([lo_ref[...].astype(_f32),
                              hi_ref[...].astype(_f32)], axis=1)
        xv = xv * degv_ref[...]
        nrm = jnp.sqrt(jnp.sum(xv * xv, axis=1, keepdims=True))
        scale = jnp.where(nrm > 0, 1.0 / jnp.maximum(nrm, 1e-30), 0.0)
        xi = 0.9 * (xv * scale) + 0.1 * x0_ref[...]
        xw = lax.dot_general(xi, wc_ref[...], (((1,), (1,)), ((), ())),
                             preferred_element_type=_f32)
        xl = jnp.maximum((1.0 - beta) * xi + beta * xw, 0.0)
        bf_ref[0] = xl[:, :HD].astype(_bf16)
        bf_ref[1] = xl[:, HD:].astype(_bf16)
        f32_ref[0] = xl[:, :HD]
        f32_ref[1] = xl[:, HD:]

    def run(pa, degv, x0, wc):
        return pl.pallas_call(
            body,
            grid=(NBLK,),
            in_specs=[
                pl.BlockSpec((BN, HD), lambda i: (i, 0)),
                pl.BlockSpec((BN, HD), lambda i: (ABLK + i, 0)),
                pl.BlockSpec((BN, 1), lambda i: (i, 0)),
                pl.BlockSpec((BN, D), lambda i: (i, 0)),
                pl.BlockSpec((D, D), lambda i: (0, 0)),
            ],
            out_specs=[
                pl.BlockSpec((2, BN, HD), lambda i: (0, i, 0)),
                pl.BlockSpec((2, BN, HD), lambda i: (0, i, 0)),
            ],
            out_shape=[
                jax.ShapeDtypeStruct((2, NPT, HD), _bf16),
                jax.ShapeDtypeStruct((2, NPT, HD), _f32),
            ],
        )(pa, pa, degv, x0, wc)

    return run


_layer_fin = [_make_layer_fin(math.log(0.5 / (i + 1) + 1.0)) for i in range(2)]


def _dense_out_body(lo_ref, hi_ref, w_ref, b_ref, out_ref):
    xb = jnp.concatenate([lo_ref[0], hi_ref[0]], axis=1)
    xw = lax.dot_general(xb, w_ref[...], (((1,), (1,)), ((), ())),
                         preferred_element_type=_f32)
    out_ref[...] = xw + b_ref[...][0]


def _dense_out(xt, wout, bout):
    return pl.pallas_call(
        _dense_out_body,
        grid=(NBLK,),
        in_specs=[
            pl.BlockSpec((1, BN, HD), lambda i: (0, i, 0)),
            pl.BlockSpec((1, BN, HD), lambda i: (1, i, 0)),
            pl.BlockSpec((FIN, D), lambda i: (0, 0)),
            pl.BlockSpec((8, FIN), lambda i: (0, 0)),
        ],
        out_specs=pl.BlockSpec((BN, FIN), lambda i: (i, 0)),
        out_shape=jax.ShapeDtypeStruct((NPT, FIN), _f32),
    )(xt, xt, wout, bout)


def kernel(x, H, vertex, edges, degV, degE, W0, b0, Wc, Wout, bout):
    vertex = vertex.astype(_i32)
    edges = edges.astype(_i32)
    zpad = jnp.zeros((PADP,), _i32)
    vg = jnp.concatenate([vertex, zpad])
    vgi = jnp.concatenate([vg, vg + NPT]).reshape(NC * IROWS, CW)
    esi = jnp.concatenate([edges, jnp.full((PADP,), M, _i32)]).reshape(IROWS, CW)
    eg = jnp.concatenate([edges, zpad])
    egi = jnp.concatenate([eg, eg + NPT]).reshape(NC * IROWS, CW)
    vsi = jnp.concatenate([vertex, jnp.full((PADP,), N, _i32)]).reshape(IROWS, CW)

    b0r = jnp.broadcast_to(b0.reshape(1, D), (8, D))
    boutr = jnp.broadcast_to(bout.reshape(1, FIN), (8, FIN))

    x0, xt = _dense_in(x, W0, b0r)
    xt0_flat = xt.reshape(TROWS, HD)

    gsel = jnp.array([0, 1, 2, 1, 2], _i32)
    ssel = jnp.array([0, 0, 1, 0, 1], _i32)
    brs = jnp.array([0, 1, 2, 1, 3], _i32)
    spread = (jnp.arange(NC * PAIRS_PAD, dtype=_i32) % TROWS).reshape(
        NC * IROWS, CW)
    gidx_all = jnp.stack([spread, vgi, egi])
    sidx_all = jnp.stack([esi, vsi])
    ones_tab = jnp.ones((TROWS, HD), _bf16)

    def loop_body(t, state):
        table, xf, cnt = state
        gidx = lax.dynamic_index_in_dim(gidx_all, gsel[t], 0, keepdims=False)
        sidx = lax.dynamic_index_in_dim(sidx_all, ssel[t], 0, keepdims=False)
        pa = _sc_pass(table, gidx, sidx)

        def b_counts(_):
            return xt0_flat, xf, lax.slice(pa, (0, 0), (MC, 16))

        def b_mk_xe(_):
            return _mk_xe(pa, cnt, degE), xf, cnt

        def b_fin0(_):
            bf, f32 = _layer_fin[0](pa, degV, x0, Wc[0])
            return bf.reshape(TROWS, HD), f32, cnt

        def b_fin1(_):
            bf, f32 = _layer_fin[1](pa, degV, x0, Wc[1])
            return bf.reshape(TROWS, HD), f32, cnt

        return lax.switch(brs[t], [b_counts, b_mk_xe, b_fin0, b_fin1], 0)

    cnt0 = jnp.zeros((MC, 16), _bf16)
    xf0 = jnp.zeros((2, NPT, HD), _f32)
    _, xf_fin, _ = lax.fori_loop(0, 5, loop_body, (ones_tab, xf0, cnt0))

    out = _dense_out(xf_fin, Wout, boutr)
    return out[:N]


# speedup vs baseline: 17.2556x; 1.0898x over previous
import functools
import math

import jax
import jax.numpy as jnp
from jax import lax
from jax.experimental import pallas as pl
from jax.experimental.pallas import tpu as pltpu
from jax.experimental.pallas import tpu_sc as plsc

N = 50000
M = 25000
E = 800000
FIN = 128
D = 64
HD = 32

NC = 2
NS = 16

CW = 64
CH = 784
PAIRS_PAD = NS * CH * CW
PADP = PAIRS_PAD - E
IROWS = PAIRS_PAD // CW
IB = 56
NBUF = 8

NPT = 50176
TROWS = NC * NPT
AROWS = 51200
MC = 25088

BN = 512
NBLK = NPT // BN
ABLK = AROWS // BN

_f32 = jnp.float32
_bf16 = jnp.bfloat16
_i32 = jnp.int32

PER_TILE = AROWS // NS


def _fill_vmem_bf16(ref, rows, value):
    def body(r, carry):
        ref[r, pl.ds(0, 32)] = jnp.full((32,), value, _bf16)
        return carry
    lax.fori_loop(0, rows, body, 0)


def _ctrl_scalar(vec, idx):
    lane = lax.iota(_i32, 16)
    return jnp.sum(jnp.where(lane == idx, vec, 0))


@functools.partial(
    pl.kernel,
    out_type=jax.ShapeDtypeStruct((NC * AROWS, HD), _bf16),
    mesh=plsc.VectorSubcoreMesh(core_axis_name="c", subcore_axis_name="s"),
    scratch_types=[
        pltpu.VMEM((IB, CW), _i32),
        pltpu.VMEM((IB, CW), _i32),
        pltpu.VMEM((16,), _i32),
    ] + [pltpu.VMEM((CW, HD), _bf16) for _ in range(2 * NBUF)] + [
        pltpu.VMEM_SHARED((AROWS, HD), _bf16),
        pltpu.SemaphoreType.DMA,
        pltpu.SemaphoreType.DMA,
        pltpu.SemaphoreType.DMA,
        pltpu.SemaphoreType.DMA,
    ],
    compiler_params=pltpu.CompilerParams(use_tc_tiling_on_sc=False,
                                         has_side_effects=True,
                                         needs_layout_passes=False),
)
def _sc_pass(table, gidx_hbm, sidx_hbm, ctrl_hbm, out, gidx, sidx, ctrl, *rest):
    bufs = [list(rest[:NBUF]), list(rest[NBUF:2 * NBUF])]
    acc = rest[2 * NBUF]
    semg = [rest[2 * NBUF + 1], rest[2 * NBUF + 2]]
    sems = [rest[2 * NBUF + 3], rest[2 * NBUF + 4]]
    c = lax.axis_index("c")
    s = lax.axis_index("s")
    r0 = bufs[0][0]
    pltpu.sync_copy(ctrl_hbm, ctrl)
    cv = ctrl[...]
    goff = _ctrl_scalar(cv, 0)
    soff = _ctrl_scalar(cv, 1)
    is_counts = _ctrl_scalar(cv, 2) == 1
    _fill_vmem_bf16(r0, CW, 0.0)
    zs = [pltpu.async_copy(r0, acc.at[pl.ds(s * PER_TILE + k * CW, CW)],
                           sems[0]) for k in range(PER_TILE // CW)]
    for d in zs:
        d.wait()
    plsc.subcore_barrier()

    NG = IB // NBUF

    @pl.when(is_counts)
    def _():
        _fill_vmem_bf16(bufs[0][0], CW, 1.0)
        _fill_vmem_bf16(bufs[1][0], CW, 1.0)

        def sc_only(u, carry):
            pltpu.sync_copy(sidx_hbm.at[pl.ds(soff + s * CH + u * IB, IB)],
                            sidx)
            pend = [None, None]
            for j in range(NG):
                st = j % 2
                if pend[st] is not None:
                    for d in pend[st]:
                        d.wait()
                pend[st] = [
                    pltpu.async_copy(bufs[st][0],
                                     acc.at[sidx.at[j * NBUF + b]],
                                     sems[st], add=True)
                    for b in range(NBUF)]
            for ps in pend:
                if ps is not None:
                    for d in ps:
                        d.wait()
            return carry

        lax.fori_loop(0, CH // IB, sc_only, 0)

    @pl.when(jnp.logical_not(is_counts))
    def _():
        def superchunk(u, carry):
            pltpu.sync_copy(
                gidx_hbm.at[pl.ds(goff + (c * NS + s) * CH + u * IB, IB)],
                gidx)
            pltpu.sync_copy(sidx_hbm.at[pl.ds(soff + s * CH + u * IB, IB)],
                            sidx)

            def fire_g(g, st):
                return [pltpu.async_copy(table.at[gidx.at[g * NBUF + b]],
                                         bufs[st][b], semg[st])
                        for b in range(NBUF)]

            def fire_s(g, st):
                return [pltpu.async_copy(bufs[st][b],
                                         acc.at[sidx.at[g * NBUF + b]],
                                         sems[st], add=True)
                        for b in range(NBUF)]

            pend_g = [None, None]
            pend_s = [None, None]
            pend_g[0] = fire_g(0, 0)
            for g in range(NG):
                st = g % 2
                ot = 1 - st
                if g + 1 < NG:
                    if pend_s[ot] is not None:
                        for d in pend_s[ot]:
                            d.wait()
                    pend_g[ot] = fire_g(g + 1, ot)
                for d in pend_g[st]:
                    d.wait()
                pend_s[st] = fire_s(g, st)
            for ps in pend_s:
                if ps is not None:
                    for d in ps:
                        d.wait()
            return carry

        lax.fori_loop(0, CH // IB, superchunk, 0)

    plsc.subcore_barrier()
    pltpu.sync_copy(acc.at[pl.ds(s * PER_TILE, PER_TILE)],
                    out.at[pl.ds(c * AROWS + s * PER_TILE, PER_TILE)])


def _dense_in_body(x_ref, w_ref, b_ref, x0_ref, xt_ref):
    xb = x_ref[...]
    xw = lax.dot_general(xb, w_ref[...], (((1,), (1,)), ((), ())),
                         preferred_element_type=_f32)
    xw = jnp.maximum(xw + b_ref[...][0], 0.0)
    x0_ref[...] = xw
    xt_ref[0] = xw[:, :HD].astype(_bf16)
    xt_ref[1] = xw[:, HD:].astype(_bf16)


def _dense_in(x, w0, b0):
    return pl.pallas_call(
        _dense_in_body,
        grid=(NBLK,),
        in_specs=[
            pl.BlockSpec((BN, FIN), lambda i: (i, 0)),
            pl.BlockSpec((D, FIN), lambda i: (0, 0)),
            pl.BlockSpec((8, D), lambda i: (0, 0)),
        ],
        out_specs=[
            pl.BlockSpec((BN, D), lambda i: (i, 0)),
            pl.BlockSpec((2, BN, HD), lambda i: (0, i, 0)),
        ],
        out_shape=[
            jax.ShapeDtypeStruct((NPT, D), _f32),
            jax.ShapeDtypeStruct((2, NPT, HD), _bf16),
        ],
    )(x, w0, b0)


def _mk_xe_body(pa_ref, cnt_ref, dege_ref, out_ref):
    cnt = cnt_ref[...][:, 0:1].astype(_f32)
    scale = dege_ref[...] / jnp.maximum(cnt, 1.0)
    out_ref[...] = (pa_ref[...].astype(_f32) * scale).astype(_bf16)


def _mk_xe(pa, cnt, dege):
    return pl.pallas_call(
        _mk_xe_body,
        grid=(NC * 49,),
        in_specs=[
            pl.BlockSpec((BN, HD), lambda i: (ABLK * (i // 49) + i % 49, 0)),
            pl.BlockSpec((BN, 16), lambda i: (i % 49, 0)),
            pl.BlockSpec((BN, 1), lambda i: (i % 49, 0)),
        ],
        out_specs=pl.BlockSpec(
            (BN, HD), lambda i: (NBLK * (i // 49) + i % 49, 0)),
        out_shape=jax.ShapeDtypeStruct((TROWS, HD), _bf16),
    )(pa, cnt, dege)


def _make_layer_fin(beta):
    def body(lo_ref, hi_ref, degv_ref, x0_ref, wc_ref, bf_ref, f32_ref):
        xv = jnp.concatenate([lo_ref[...].astype(_f32),
                              hi_ref[...].astype(_f32)], axis=1)
        xv = xv * degv_ref[...]
        nrm = jnp.sqrt(jnp.sum(xv * xv, axis=1, keepdims=True))
        scale = jnp.where(nrm > 0, 1.0 / jnp.maximum(nrm, 1e-30), 0.0)
        xi = 0.9 * (xv * scale) + 0.1 * x0_ref[...]
        xw = lax.dot_general(xi, wc_ref[...], (((1,), (1,)), ((), ())),
                             preferred_element_type=_f32)
        xl = jnp.maximum((1.0 - beta) * xi + beta * xw, 0.0)
        bf_ref[0] = xl[:, :HD].astype(_bf16)
        bf_ref[1] = xl[:, HD:].astype(_bf16)
        f32_ref[0] = xl[:, :HD]
        f32_ref[1] = xl[:, HD:]

    def run(pa, degv, x0, wc):
        return pl.pallas_call(
            body,
            grid=(NBLK,),
            in_specs=[
                pl.BlockSpec((BN, HD), lambda i: (i, 0)),
                pl.BlockSpec((BN, HD), lambda i: (ABLK + i, 0)),
                pl.BlockSpec((BN, 1), lambda i: (i, 0)),
                pl.BlockSpec((BN, D), lambda i: (i, 0)),
                pl.BlockSpec((D, D), lambda i: (0, 0)),
            ],
            out_specs=[
                pl.BlockSpec((2, BN, HD), lambda i: (0, i, 0)),
                pl.BlockSpec((2, BN, HD), lambda i: (0, i, 0)),
            ],
            out_shape=[
                jax.ShapeDtypeStruct((2, NPT, HD), _bf16),
                jax.ShapeDtypeStruct((2, NPT, HD), _f32),
            ],
        )(pa, pa, degv, x0, wc)

    return run


_layer_fin = [_make_layer_fin(math.log(0.5 / (i + 1) + 1.0)) for i in range(2)]


def _dense_out_body(lo_ref, hi_ref, w_ref, b_ref, out_ref):
    xb = jnp.concatenate([lo_ref[0], hi_ref[0]], axis=1)
    xw = lax.dot_general(xb, w_ref[...], (((1,), (1,)), ((), ())),
                         preferred_element_type=_f32)
    out_ref[...] = xw + b_ref[...][0]


def _dense_out(xt, wout, bout):
    return pl.pallas_call(
        _dense_out_body,
        grid=(NBLK,),
        in_specs=[
            pl.BlockSpec((1, BN, HD), lambda i: (0, i, 0)),
            pl.BlockSpec((1, BN, HD), lambda i: (1, i, 0)),
            pl.BlockSpec((FIN, D), lambda i: (0, 0)),
            pl.BlockSpec((8, FIN), lambda i: (0, 0)),
        ],
        out_specs=pl.BlockSpec((BN, FIN), lambda i: (i, 0)),
        out_shape=jax.ShapeDtypeStruct((NPT, FIN), _f32),
    )(xt, xt, wout, bout)


def kernel(x, H, vertex, edges, degV, degE, W0, b0, Wc, Wout, bout):
    vertex = vertex.astype(_i32)
    edges = edges.astype(_i32)
    zpad = jnp.zeros((PADP,), _i32)
    vg = jnp.concatenate([vertex, zpad])
    vgi = jnp.concatenate([vg, vg + NPT]).reshape(NC * IROWS, CW)
    esi = jnp.concatenate([edges, jnp.full((PADP,), M, _i32)]).reshape(IROWS, CW)
    eg = jnp.concatenate([edges, zpad])
    egi = jnp.concatenate([eg, eg + NPT]).reshape(NC * IROWS, CW)
    vsi = jnp.concatenate([vertex, jnp.full((PADP,), N, _i32)]).reshape(IROWS, CW)

    b0r = jnp.broadcast_to(b0.reshape(1, D), (8, D))
    boutr = jnp.broadcast_to(bout.reshape(1, FIN), (8, FIN))

    x0, xt = _dense_in(x, W0, b0r)
    xt0_flat = xt.reshape(TROWS, HD)

    brs = jnp.array([0, 1, 2, 1, 3], _i32)
    gidx_all = jnp.concatenate([vgi, egi])
    sidx_all = jnp.concatenate([esi, vsi])
    ctrl_all = jnp.zeros((5, 16), _i32)
    ctrl_all = ctrl_all.at[0, 2].set(1)
    ctrl_all = ctrl_all.at[2, 0].set(NC * IROWS).at[2, 1].set(IROWS)
    ctrl_all = ctrl_all.at[4, 0].set(NC * IROWS).at[4, 1].set(IROWS)

    def loop_body(t, state):
        table, xf, cnt = state
        ctrl = lax.dynamic_index_in_dim(ctrl_all, t, 0, keepdims=False)
        pa = _sc_pass(table, gidx_all, sidx_all, ctrl)

        def b_counts(_):
            return xt0_flat, xf, lax.slice(pa, (0, 0), (MC, 16))

        def b_mk_xe(_):
            return _mk_xe(pa, cnt, degE), xf, cnt

        def b_fin0(_):
            bf, f32 = _layer_fin[0](pa, degV, x0, Wc[0])
            return bf.reshape(TROWS, HD), f32, cnt

        def b_fin1(_):
            bf, f32 = _layer_fin[1](pa, degV, x0, Wc[1])
            return bf.reshape(TROWS, HD), f32, cnt

        return lax.switch(brs[t], [b_counts, b_mk_xe, b_fin0, b_fin1], 0)

    cnt0 = jnp.zeros((MC, 16), _bf16)
    xf0 = jnp.zeros((2, NPT, HD), _f32)
    _, xf_fin, _ = lax.fori_loop(0, 5, loop_body, (xt0_flat, xf0, cnt0))

    out = _dense_out(xf_fin, Wout, boutr)
    return out[:N]


# speedup vs baseline: 18.3470x; 1.0632x over previous
import functools
import math

import jax
import jax.numpy as jnp
from jax import lax
from jax.experimental import pallas as pl
from jax.experimental.pallas import tpu as pltpu
from jax.experimental.pallas import tpu_sc as plsc

N = 50000
M = 25000
E = 800000
FIN = 128
D = 64
HD = 32

NC = 2
NS = 16

CW = 64
CH = 784
PAIRS_PAD = NS * CH * CW
PADP = PAIRS_PAD - E
IROWS = PAIRS_PAD // CW
IB = 56
NBUF = 8

NPT = 50176
TROWS = NC * NPT
AROWS = 51200
MC = 25088
CB = 25600

BN = 512
NBLK = NPT // BN
ABLK = AROWS // BN

_f32 = jnp.float32
_bf16 = jnp.bfloat16
_i32 = jnp.int32

PER_TILE = AROWS // NS


def _fill_vmem_bf16(ref, rows, value):
    def body(r, carry):
        ref[r, pl.ds(0, 32)] = jnp.full((32,), value, _bf16)
        return carry
    lax.fori_loop(0, rows, body, 0)


def _ctrl_scalar(vec, idx):
    lane = lax.iota(_i32, 16)
    return jnp.sum(jnp.where(lane == idx, vec, 0))


@functools.partial(
    pl.kernel,
    out_type=jax.ShapeDtypeStruct((NC * AROWS, HD), _bf16),
    mesh=plsc.VectorSubcoreMesh(core_axis_name="c", subcore_axis_name="s"),
    scratch_types=[
        pltpu.VMEM((IB, CW), _i32),
        pltpu.VMEM((IB, CW), _i32),
        pltpu.VMEM((16,), _i32),
    ] + [pltpu.VMEM((CW, HD), _bf16) for _ in range(2 * NBUF)] + [
        pltpu.VMEM_SHARED((AROWS, HD), _bf16),
        pltpu.SemaphoreType.DMA,
        pltpu.SemaphoreType.DMA,
        pltpu.SemaphoreType.DMA,
        pltpu.SemaphoreType.DMA,
    ],
    compiler_params=pltpu.CompilerParams(use_tc_tiling_on_sc=False,
                                         has_side_effects=True,
                                         needs_layout_passes=False),
)
def _sc_pass(table, gidx_hbm, sidx_hbm, ctrl_hbm, out, gidx, sidx, ctrl, *rest):
    bufs = [list(rest[:NBUF]), list(rest[NBUF:2 * NBUF])]
    acc = rest[2 * NBUF]
    semg = [rest[2 * NBUF + 1], rest[2 * NBUF + 2]]
    sems = [rest[2 * NBUF + 3], rest[2 * NBUF + 4]]
    c = lax.axis_index("c")
    s = lax.axis_index("s")
    r0 = bufs[0][0]
    pltpu.sync_copy(ctrl_hbm, ctrl)
    cv = ctrl[...]
    goff = _ctrl_scalar(cv, 0)
    soff = _ctrl_scalar(cv, 1)
    do_count = _ctrl_scalar(cv, 2) == 1
    _fill_vmem_bf16(r0, CW, 0.0)
    zs = [pltpu.async_copy(r0, acc.at[pl.ds(s * PER_TILE + k * CW, CW)],
                           sems[0]) for k in range(PER_TILE // CW)]
    for d in zs:
        d.wait()
    plsc.subcore_barrier()

    NG = IB // NBUF

    def superchunk(u, carry):
        pltpu.sync_copy(
            gidx_hbm.at[pl.ds(goff + (c * NS + s) * CH + u * IB, IB)],
            gidx)
        pltpu.sync_copy(sidx_hbm.at[pl.ds(soff + s * CH + u * IB, IB)],
                        sidx)

        def fire_g(g, st):
            return [pltpu.async_copy(table.at[gidx.at[g * NBUF + b]],
                                     bufs[st][b], semg[st])
                    for b in range(NBUF)]

        def fire_s(g, st):
            return [pltpu.async_copy(bufs[st][b],
                                     acc.at[sidx.at[g * NBUF + b]],
                                     sems[st], add=True)
                    for b in range(NBUF)]

        pend_g = [None, None]
        pend_s = [None, None]
        pend_g[0] = fire_g(0, 0)
        for g in range(NG):
            st = g % 2
            ot = 1 - st
            if g + 1 < NG:
                if pend_s[ot] is not None:
                    for d in pend_s[ot]:
                        d.wait()
                pend_g[ot] = fire_g(g + 1, ot)
            for d in pend_g[st]:
                d.wait()
            pend_s[st] = fire_s(g, st)
        for ps in pend_s:
            if ps is not None:
                for d in ps:
                    d.wait()
        return carry

    lax.fori_loop(0, CH // IB, superchunk, 0)

    @pl.when(do_count)
    def _():
        ones = bufs[0][0]
        _fill_vmem_bf16(ones, CW, 1.0)

        def cchunk(u, carry):
            pltpu.sync_copy(
                sidx_hbm.at[pl.ds(2 * IROWS + s * CH + u * IB, IB)], gidx)
            pend = [None, None]
            for j in range(NG):
                st = j % 2
                if pend[st] is not None:
                    for d in pend[st]:
                        d.wait()
                pend[st] = [
                    pltpu.async_copy(ones, acc.at[gidx.at[j * NBUF + b]],
                                     sems[st], add=True)
                    for b in range(NBUF)]
            for ps in pend:
                if ps is not None:
                    for d in ps:
                        d.wait()
            return carry

        lax.fori_loop(0, CH // IB, cchunk, 0)

    plsc.subcore_barrier()
    pltpu.sync_copy(acc.at[pl.ds(s * PER_TILE, PER_TILE)],
                    out.at[pl.ds(c * AROWS + s * PER_TILE, PER_TILE)])


def _dense_in_body(x_ref, w_ref, b_ref, x0_ref, xt_ref):
    xb = x_ref[...]
    xw = lax.dot_general(xb, w_ref[...], (((1,), (1,)), ((), ())),
                         preferred_element_type=_f32)
    xw = jnp.maximum(xw + b_ref[...][0], 0.0)
    x0_ref[...] = xw
    xt_ref[0] = xw[:, :HD].astype(_bf16)
    xt_ref[1] = xw[:, HD:].astype(_bf16)


def _dense_in(x, w0, b0):
    return pl.pallas_call(
        _dense_in_body,
        grid=(NBLK,),
        in_specs=[
            pl.BlockSpec((BN, FIN), lambda i: (i, 0)),
            pl.BlockSpec((D, FIN), lambda i: (0, 0)),
            pl.BlockSpec((8, D), lambda i: (0, 0)),
        ],
        out_specs=[
            pl.BlockSpec((BN, D), lambda i: (i, 0)),
            pl.BlockSpec((2, BN, HD), lambda i: (0, i, 0)),
        ],
        out_shape=[
            jax.ShapeDtypeStruct((NPT, D), _f32),
            jax.ShapeDtypeStruct((2, NPT, HD), _bf16),
        ],
    )(x, w0, b0)


def _mk_xe_body(pa_ref, cnt_ref, dege_ref, out_ref):
    cnt = cnt_ref[...][:, 0:1].astype(_f32)
    scale = dege_ref[...] / jnp.maximum(cnt, 1.0)
    out_ref[...] = (pa_ref[...].astype(_f32) * scale).astype(_bf16)


def _mk_xe(pa, cnt, dege):
    return pl.pallas_call(
        _mk_xe_body,
        grid=(NC * 49,),
        in_specs=[
            pl.BlockSpec((BN, HD), lambda i: (ABLK * (i // 49) + i % 49, 0)),
            pl.BlockSpec((BN, 16), lambda i: (i % 49, 0)),
            pl.BlockSpec((BN, 1), lambda i: (i % 49, 0)),
        ],
        out_specs=pl.BlockSpec(
            (BN, HD), lambda i: (NBLK * (i // 49) + i % 49, 0)),
        out_shape=jax.ShapeDtypeStruct((TROWS, HD), _bf16),
    )(pa, cnt, dege)


def _make_layer_fin(beta):
    def body(lo_ref, hi_ref, degv_ref, x0_ref, wc_ref, bf_ref, f32_ref):
        xv = jnp.concatenate([lo_ref[...].astype(_f32),
                              hi_ref[...].astype(_f32)], axis=1)
        xv = xv * degv_ref[...]
        nrm = jnp.sqrt(jnp.sum(xv * xv, axis=1, keepdims=True))
        scale = jnp.where(nrm > 0, 1.0 / jnp.maximum(nrm, 1e-30), 0.0)
        xi = 0.9 * (xv * scale) + 0.1 * x0_ref[...]
        xw = lax.dot_general(xi, wc_ref[...], (((1,), (1,)), ((), ())),
                             preferred_element_type=_f32)
        xl = jnp.maximum((1.0 - beta) * xi + beta * xw, 0.0)
        bf_ref[0] = xl[:, :HD].astype(_bf16)
        bf_ref[1] = xl[:, HD:].astype(_bf16)
        f32_ref[0] = xl[:, :HD]
        f32_ref[1] = xl[:, HD:]

    def run(pa, degv, x0, wc):
        return pl.pallas_call(
            body,
            grid=(NBLK,),
            in_specs=[
                pl.BlockSpec((BN, HD), lambda i: (i, 0)),
                pl.BlockSpec((BN, HD), lambda i: (ABLK + i, 0)),
                pl.BlockSpec((BN, 1), lambda i: (i, 0)),
                pl.BlockSpec((BN, D), lambda i: (i, 0)),
                pl.BlockSpec((D, D), lambda i: (0, 0)),
            ],
            out_specs=[
                pl.BlockSpec((2, BN, HD), lambda i: (0, i, 0)),
                pl.BlockSpec((2, BN, HD), lambda i: (0, i, 0)),
            ],
            out_shape=[
                jax.ShapeDtypeStruct((2, NPT, HD), _bf16),
                jax.ShapeDtypeStruct((2, NPT, HD), _f32),
            ],
        )(pa, pa, degv, x0, wc)

    return run


_layer_fin = [_make_layer_fin(math.log(0.5 / (i + 1) + 1.0)) for i in range(2)]


def _dense_out_body(lo_ref, hi_ref, w_ref, b_ref, out_ref):
    xb = jnp.concatenate([lo_ref[0], hi_ref[0]], axis=1)
    xw = lax.dot_general(xb, w_ref[...], (((1,), (1,)), ((), ())),
                         preferred_element_type=_f32)
    out_ref[...] = xw + b_ref[...][0]


def _dense_out(xt, wout, bout):
    return pl.pallas_call(
        _dense_out_body,
        grid=(NBLK,),
        in_specs=[
            pl.BlockSpec((1, BN, HD), lambda i: (0, i, 0)),
            pl.BlockSpec((1, BN, HD), lambda i: (1, i, 0)),
            pl.BlockSpec((FIN, D), lambda i: (0, 0)),
            pl.BlockSpec((8, FIN), lambda i: (0, 0)),
        ],
        out_specs=pl.BlockSpec((BN, FIN), lambda i: (i, 0)),
        out_shape=jax.ShapeDtypeStruct((NPT, FIN), _f32),
    )(xt, xt, wout, bout)


def kernel(x, H, vertex, edges, degV, degE, W0, b0, Wc, Wout, bout):
    vertex = vertex.astype(_i32)
    edges = edges.astype(_i32)
    zpad = jnp.zeros((PADP,), _i32)
    vg = jnp.concatenate([vertex, zpad])
    vgi = jnp.concatenate([vg, vg + NPT]).reshape(NC * IROWS, CW)
    esi = jnp.concatenate([edges, jnp.full((PADP,), M, _i32)]).reshape(IROWS, CW)
    eg = jnp.concatenate([edges, zpad])
    egi = jnp.concatenate([eg, eg + NPT]).reshape(NC * IROWS, CW)
    vsi = jnp.concatenate([vertex, jnp.full((PADP,), N, _i32)]).reshape(IROWS, CW)

    b0r = jnp.broadcast_to(b0.reshape(1, D), (8, D))
    boutr = jnp.broadcast_to(bout.reshape(1, FIN), (8, FIN))

    x0, xt = _dense_in(x, W0, b0r)
    xt0_flat = xt.reshape(TROWS, HD)

    gidx_all = jnp.concatenate([vgi, egi])
    sidx_all = jnp.concatenate([esi, vsi, esi + CB])
    ctrl_all = jnp.zeros((4, 16), _i32)
    ctrl_all = ctrl_all.at[0, 2].set(1)
    ctrl_all = ctrl_all.at[1, 0].set(NC * IROWS).at[1, 1].set(IROWS)
    ctrl_all = ctrl_all.at[3, 0].set(NC * IROWS).at[3, 1].set(IROWS)

    def loop_body(t, state):
        table, xf, cnt = state
        ctrl = lax.dynamic_index_in_dim(ctrl_all, t, 0, keepdims=False)
        pa = _sc_pass(table, gidx_all, sidx_all, ctrl)

        def b_mk_xe0(_):
            cnt2 = lax.slice(pa, (CB, 0), (CB + MC, 16))
            return _mk_xe(pa, cnt2, degE), xf, cnt2

        def b_mk_xe1(_):
            return _mk_xe(pa, cnt, degE), xf, cnt

        def b_fin0(_):
            bf, f32 = _layer_fin[0](pa, degV, x0, Wc[0])
            return bf.reshape(TROWS, HD), f32, cnt

        def b_fin1(_):
            bf, f32 = _layer_fin[1](pa, degV, x0, Wc[1])
            return bf.reshape(TROWS, HD), f32, cnt

        return lax.switch(t, [b_mk_xe0, b_fin0, b_mk_xe1, b_fin1], 0)

    cnt0 = jnp.zeros((MC, 16), _bf16)
    xf0 = jnp.zeros((2, NPT, HD), _f32)
    _, xf_fin, _ = lax.fori_loop(0, 4, loop_body, (xt0_flat, xf0, cnt0))

    out = _dense_out(xf_fin, Wout, boutr)
    return out[:N]
